# Initial kernel scaffold; baseline (speedup 1.0000x reference)
import functools

import jax
import jax.numpy as jnp
from jax import lax
from jax.experimental import pallas as pl
from jax.experimental.pallas import tpu as pltpu
from jax.experimental.pallas import tpu_sc as plsc

NC = 2
NS = 16
CW = 16


@functools.lru_cache(maxsize=None)
def _make_sc_segsum(n_nodes: int, h_dim: int, n_edges: int):
  NW = NC * NS
  epw = n_edges // NW
  K = 80
  nch = epw // K
  assert epw % K == 0 and n_nodes % NS == 0
  rps = n_nodes // NS

  mesh = plsc.VectorSubcoreMesh(core_axis_name="c", subcore_axis_name="s")

  @functools.partial(
      pl.kernel,
      out_type=(
          jax.ShapeDtypeStruct((NC, n_nodes, h_dim), jnp.float32),
          jax.ShapeDtypeStruct((NC, n_nodes, CW), jnp.float32),
      ),
      mesh=mesh,
      scratch_types=[
          pltpu.VMEM_SHARED((n_nodes, h_dim), jnp.float32),
          pltpu.VMEM_SHARED((n_nodes, CW), jnp.float32),
          pltpu.VMEM((K,), jnp.int32),
          pltpu.VMEM((K,), jnp.int32),
          pltpu.VMEM((K, h_dim), jnp.float32),
          pltpu.VMEM((K, CW), jnp.float32),
          pltpu.SemaphoreType.DMA,
      ],
  )
  def seg(h_hbm, src_hbm, dst_hbm, z64_hbm, z16_hbm, ones_hbm,
          agg_out, cnt_out, agg_s, cnt_s, src_v, dst_v, rows_v, ones_v, sem):
    c = lax.axis_index("c")
    s = lax.axis_index("s")
    r0 = s * rps
    pltpu.sync_copy(z64_hbm.at[pl.ds(r0, rps)], agg_s.at[pl.ds(r0, rps)])
    pltpu.sync_copy(z16_hbm.at[pl.ds(r0, rps)], cnt_s.at[pl.ds(r0, rps)])
    pltpu.sync_copy(ones_hbm, ones_v)
    plsc.subcore_barrier()
    base = (c * NS + s) * epw

    def body(i, carry):
      off = base + i * K
      pltpu.sync_copy(src_hbm.at[pl.ds(off, K)], src_v)
      pltpu.sync_copy(dst_hbm.at[pl.ds(off, K)], dst_v)
      pltpu.async_copy(h_hbm.at[src_v], rows_v, sem).wait()
      pltpu.sync_copy(rows_v, agg_s.at[dst_v], add=True)
      pltpu.sync_copy(ones_v, cnt_s.at[dst_v], add=True)
      return carry

    lax.fori_loop(0, nch, body, 0)
    plsc.subcore_barrier()
    pltpu.sync_copy(agg_s.at[pl.ds(r0, rps)], agg_out.at[c, pl.ds(r0, rps)])
    pltpu.sync_copy(cnt_s.at[pl.ds(r0, rps)], cnt_out.at[c, pl.ds(r0, rps)])

  return seg


def _input_proj(x, w, b):
  n, d = x.shape
  h = w.shape[1]
  rb = 2000

  def body(x_ref, w_ref, b_ref, o_ref):
    acc = jnp.dot(x_ref[...], w_ref[...], preferred_element_type=jnp.float32)
    o_ref[...] = jnp.maximum(acc + b_ref[...], 0.0)

  return pl.pallas_call(
      body,
      grid=(n // rb,),
      in_specs=[
          pl.BlockSpec((rb, d), lambda i: (i, 0)),
          pl.BlockSpec((d, h), lambda i: (0, 0)),
          pl.BlockSpec((1, h), lambda i: (0, 0)),
      ],
      out_specs=pl.BlockSpec((rb, h), lambda i: (i, 0)),
      out_shape=jax.ShapeDtypeStruct((n, h), jnp.float32),
  )(x, w, b.reshape(1, h))


def _layer_combine(agg2, cnt2, h, wl, bl, wr):
  n, hd = h.shape
  rb = 2000

  def body(a_ref, c_ref, h_ref, wl_ref, bl_ref, wr_ref, o_ref):
    a = a_ref[0] + a_ref[1]
    cg = c_ref[0, :, :1] + c_ref[1, :, :1]
    mean = a * (1.0 / jnp.maximum(cg, 1.0))
    acc = jnp.dot(mean, wl_ref[...], preferred_element_type=jnp.float32)
    acc = acc + jnp.dot(h_ref[...], wr_ref[...],
                        preferred_element_type=jnp.float32)
    o_ref[...] = jnp.maximum(acc + bl_ref[...], 0.0)

  return pl.pallas_call(
      body,
      grid=(n // rb,),
      in_specs=[
          pl.BlockSpec((NC, rb, hd), lambda i: (0, i, 0)),
          pl.BlockSpec((NC, rb, CW), lambda i: (0, i, 0)),
          pl.BlockSpec((rb, hd), lambda i: (i, 0)),
          pl.BlockSpec((hd, hd), lambda i: (0, 0)),
          pl.BlockSpec((1, hd), lambda i: (0, 0)),
          pl.BlockSpec((hd, hd), lambda i: (0, 0)),
      ],
      out_specs=pl.BlockSpec((rb, hd), lambda i: (i, 0)),
      out_shape=jax.ShapeDtypeStruct((n, hd), jnp.float32),
  )(agg2, cnt2, h, wl, bl.reshape(1, hd), wr)


def _pool_mlp(h, batch_col, gf, w1, b1, w2, b2):
  n, hd = h.shape
  g = gf.shape[0]
  gfd = gf.shape[1]
  rb = 1000
  nb = n // rb
  w1a = w1[:hd]
  w1b = w1[hd:2 * hd]
  w1c = w1[2 * hd:]

  def body(h_ref, b_ref, gf_ref, w1a_ref, w1b_ref, w1c_ref, b1_ref,
           w2_ref, b2_ref, o_ref, sums, cnts, maxs):
    i = pl.program_id(0)

    @pl.when(i == 0)
    def _init():
      sums[...] = jnp.zeros_like(sums)
      cnts[...] = jnp.zeros_like(cnts)
      maxs[...] = jnp.zeros_like(maxs)

    hb = h_ref[...]
    gid = lax.broadcasted_iota(jnp.int32, (rb, g), 1)
    mask = (b_ref[...] == gid).astype(jnp.float32)
    dn = (((0,), (0,)), ((), ()))
    sums[...] += lax.dot_general(mask, hb, dn,
                                 preferred_element_type=jnp.float32)
    cnts[...] += lax.dot_general(mask, jnp.ones_like(hb), dn,
                                 preferred_element_type=jnp.float32)
    for gc in range(0, g, 8):
      sel = mask[:, gc:gc + 8]
      cand = jnp.max(hb[:, None, :] * sel[:, :, None], axis=0)
      maxs[gc:gc + 8, :] = jnp.maximum(maxs[gc:gc + 8, :], cand)

    @pl.when(i == nb - 1)
    def _final():
      meanp = sums[...] / jnp.maximum(cnts[...], 1.0)
      z1 = jnp.dot(meanp, w1a_ref[...], preferred_element_type=jnp.float32)
      z1 = z1 + jnp.dot(maxs[...], w1b_ref[...],
                        preferred_element_type=jnp.float32)
      z1 = z1 + jnp.dot(gf_ref[...], w1c_ref[...],
                        preferred_element_type=jnp.float32)
      z1 = jnp.maximum(z1 + b1_ref[...], 0.0)
      z2 = jnp.dot(z1, w2_ref[...], preferred_element_type=jnp.float32)
      o_ref[...] = jnp.maximum(z2 + b2_ref[...], 0.0)

  return pl.pallas_call(
      body,
      grid=(nb,),
      in_specs=[
          pl.BlockSpec((rb, hd), lambda i: (i, 0)),
          pl.BlockSpec((rb, 1), lambda i: (i, 0)),
          pl.BlockSpec((g, gfd), lambda i: (0, 0)),
          pl.BlockSpec((hd, hd), lambda i: (0, 0)),
          pl.BlockSpec((hd, hd), lambda i: (0, 0)),
          pl.BlockSpec((gfd, hd), lambda i: (0, 0)),
          pl.BlockSpec((1, hd), lambda i: (0, 0)),
          pl.BlockSpec((hd, hd), lambda i: (0, 0)),
          pl.BlockSpec((1, hd), lambda i: (0, 0)),
      ],
      out_specs=pl.BlockSpec((g, hd), lambda i: (0, 0)),
      out_shape=jax.ShapeDtypeStruct((g, hd), jnp.float32),
      scratch_shapes=[
          pltpu.VMEM((g, hd), jnp.float32),
          pltpu.VMEM((g, hd), jnp.float32),
          pltpu.VMEM((g, hd), jnp.float32),
      ],
  )(h, batch_col, gf, w1a, w1b, w1c, b1.reshape(1, hd), w2, b2.reshape(1, hd))


def kernel(x, edge_index, batch, graph_feat, W_in, b_in,
           Wl0, bl0, Wr0, Wl1, bl1, Wr1, Wl2, bl2, Wr2,
           W1, b1, W2, b2):
  n = x.shape[0]
  hd = W_in.shape[1]
  e = edge_index.shape[1]
  src = edge_index[0]
  dst = edge_index[1]

  z64 = jnp.zeros((n, hd), jnp.float32)
  z16 = jnp.zeros((n, CW), jnp.float32)
  ones16 = jnp.ones((80, CW), jnp.float32)

  seg = _make_sc_segsum(n, hd, e)

  h = _input_proj(x, W_in, b_in)
  for (wl, bl, wr) in ((Wl0, bl0, Wr0), (Wl1, bl1, Wr1), (Wl2, bl2, Wr2)):
    agg2, cnt2 = seg(h, src, dst, z64, z16, ones16)
    h = _layer_combine(agg2, cnt2, h, wl, bl, wr)

  return _pool_mlp(h, batch.reshape(n, 1), graph_feat, W1, b1, W2, b2)


# speedup vs baseline: 4.5726x; 4.5726x over previous
import functools

import jax
import jax.numpy as jnp
from jax import lax
from jax.experimental import pallas as pl
from jax.experimental.pallas import tpu as pltpu
from jax.experimental.pallas import tpu_sc as plsc

NC = 2
NS = 16
CW = 16


@functools.lru_cache(maxsize=None)
def _make_sc_segsum(n_nodes: int, h_dim: int, n_edges: int):
  NW = NC * NS
  epw = n_edges // NW
  K = 80
  nch = epw // K
  assert epw % K == 0
  rps = (n_nodes // NS) // 8 * 8
  tail = n_nodes - NS * rps

  mesh = plsc.VectorSubcoreMesh(core_axis_name="c", subcore_axis_name="s")

  @functools.partial(
      pl.kernel,
      out_type=(
          jax.ShapeDtypeStruct((NC, n_nodes, h_dim), jnp.float32),
          jax.ShapeDtypeStruct((NC, n_nodes, CW), jnp.float32),
      ),
      mesh=mesh,
      compiler_params=pltpu.CompilerParams(use_tc_tiling_on_sc=False),
      scratch_types=[
          pltpu.VMEM_SHARED((n_nodes, h_dim), jnp.float32),
          pltpu.VMEM_SHARED((n_nodes, CW), jnp.float32),
          pltpu.VMEM((K,), jnp.int32),
          pltpu.VMEM((K,), jnp.int32),
          pltpu.VMEM((K, h_dim), jnp.float32),
          pltpu.VMEM((K, CW), jnp.float32),
          pltpu.SemaphoreType.DMA,
      ],
  )
  def seg(h_hbm, src_hbm, dst_hbm, z64_hbm, z16_hbm, ones_hbm,
          agg_out, cnt_out, agg_s, cnt_s, src_v, dst_v, rows_v, ones_v, sem):
    c = lax.axis_index("c")
    s = lax.axis_index("s")
    r0 = s * rps
    pltpu.sync_copy(z64_hbm.at[pl.ds(r0, rps)], agg_s.at[pl.ds(r0, rps)])
    pltpu.sync_copy(z16_hbm.at[pl.ds(r0, rps)], cnt_s.at[pl.ds(r0, rps)])
    if tail:
      @pl.when(s == 0)
      def _tail_init():
        t0 = NS * rps
        pltpu.sync_copy(z64_hbm.at[pl.ds(t0, tail)], agg_s.at[pl.ds(t0, tail)])
        pltpu.sync_copy(z16_hbm.at[pl.ds(t0, tail)], cnt_s.at[pl.ds(t0, tail)])
    pltpu.sync_copy(ones_hbm, ones_v)
    plsc.subcore_barrier()
    base = (c * NS + s) * epw

    def body(i, carry):
      off = base + i * K
      pltpu.sync_copy(src_hbm.at[pl.ds(off, K)], src_v)
      pltpu.sync_copy(dst_hbm.at[pl.ds(off, K)], dst_v)
      pltpu.async_copy(h_hbm.at[src_v], rows_v, sem).wait()
      pltpu.sync_copy(rows_v, agg_s.at[dst_v], add=True)
      pltpu.sync_copy(ones_v, cnt_s.at[dst_v], add=True)
      return carry

    lax.fori_loop(0, nch, body, 0)
    plsc.subcore_barrier()
    pltpu.sync_copy(agg_s.at[pl.ds(r0, rps)], agg_out.at[c, pl.ds(r0, rps)])
    pltpu.sync_copy(cnt_s.at[pl.ds(r0, rps)], cnt_out.at[c, pl.ds(r0, rps)])
    if tail:
      @pl.when(s == 0)
      def _tail_out():
        t0 = NS * rps
        pltpu.sync_copy(agg_s.at[pl.ds(t0, tail)],
                        agg_out.at[c, pl.ds(t0, tail)])
        pltpu.sync_copy(cnt_s.at[pl.ds(t0, tail)],
                        cnt_out.at[c, pl.ds(t0, tail)])

  return seg


def _input_proj(x, w, b):
  n, d = x.shape
  h = w.shape[1]
  rb = 2000

  def body(x_ref, w_ref, b_ref, o_ref):
    acc = jnp.dot(x_ref[...], w_ref[...], preferred_element_type=jnp.float32)
    o_ref[...] = jnp.maximum(acc + b_ref[...], 0.0)

  return pl.pallas_call(
      body,
      grid=(n // rb,),
      in_specs=[
          pl.BlockSpec((rb, d), lambda i: (i, 0)),
          pl.BlockSpec((d, h), lambda i: (0, 0)),
          pl.BlockSpec((1, h), lambda i: (0, 0)),
      ],
      out_specs=pl.BlockSpec((rb, h), lambda i: (i, 0)),
      out_shape=jax.ShapeDtypeStruct((n, h), jnp.float32),
  )(x, w, b.reshape(1, h))


def _layer_combine(agg2, cnt2, h, wl, bl, wr):
  n, hd = h.shape
  rb = 2000

  def body(a_ref, c_ref, h_ref, wl_ref, bl_ref, wr_ref, o_ref):
    a = a_ref[0] + a_ref[1]
    cg = c_ref[0, :, :1] + c_ref[1, :, :1]
    mean = a * (1.0 / jnp.maximum(cg, 1.0))
    acc = jnp.dot(mean, wl_ref[...], preferred_element_type=jnp.float32)
    acc = acc + jnp.dot(h_ref[...], wr_ref[...],
                        preferred_element_type=jnp.float32)
    o_ref[...] = jnp.maximum(acc + bl_ref[...], 0.0)

  return pl.pallas_call(
      body,
      grid=(n // rb,),
      in_specs=[
          pl.BlockSpec((NC, rb, hd), lambda i: (0, i, 0)),
          pl.BlockSpec((NC, rb, CW), lambda i: (0, i, 0)),
          pl.BlockSpec((rb, hd), lambda i: (i, 0)),
          pl.BlockSpec((hd, hd), lambda i: (0, 0)),
          pl.BlockSpec((1, hd), lambda i: (0, 0)),
          pl.BlockSpec((hd, hd), lambda i: (0, 0)),
      ],
      out_specs=pl.BlockSpec((rb, hd), lambda i: (i, 0)),
      out_shape=jax.ShapeDtypeStruct((n, hd), jnp.float32),
  )(agg2, cnt2, h, wl, bl.reshape(1, hd), wr)


def _pool_mlp(h, batch_col, gf, w1, b1, w2, b2):
  n, hd = h.shape
  g = gf.shape[0]
  gfd = gf.shape[1]
  rb = 1000
  nb = n // rb
  w1a = w1[:hd]
  w1b = w1[hd:2 * hd]
  w1c = w1[2 * hd:]

  def body(h_ref, b_ref, gf_ref, w1a_ref, w1b_ref, w1c_ref, b1_ref,
           w2_ref, b2_ref, o_ref, sums, cnts, maxs):
    i = pl.program_id(0)

    @pl.when(i == 0)
    def _init():
      sums[...] = jnp.zeros_like(sums)
      cnts[...] = jnp.zeros_like(cnts)
      maxs[...] = jnp.zeros_like(maxs)

    hb = h_ref[...]
    gid = lax.broadcasted_iota(jnp.int32, (rb, g), 1)
    mask = (b_ref[...] == gid).astype(jnp.float32)
    dn = (((0,), (0,)), ((), ()))
    sums[...] += lax.dot_general(mask, hb, dn,
                                 preferred_element_type=jnp.float32)
    cnts[...] += lax.dot_general(mask, jnp.ones_like(hb), dn,
                                 preferred_element_type=jnp.float32)
    for gc in range(0, g, 8):
      sel = mask[:, gc:gc + 8]
      cand = jnp.max(hb[:, None, :] * sel[:, :, None], axis=0)
      maxs[gc:gc + 8, :] = jnp.maximum(maxs[gc:gc + 8, :], cand)

    @pl.when(i == nb - 1)
    def _final():
      meanp = sums[...] / jnp.maximum(cnts[...], 1.0)
      z1 = jnp.dot(meanp, w1a_ref[...], preferred_element_type=jnp.float32)
      z1 = z1 + jnp.dot(maxs[...], w1b_ref[...],
                        preferred_element_type=jnp.float32)
      z1 = z1 + jnp.dot(gf_ref[...], w1c_ref[...],
                        preferred_element_type=jnp.float32)
      z1 = jnp.maximum(z1 + b1_ref[...], 0.0)
      z2 = jnp.dot(z1, w2_ref[...], preferred_element_type=jnp.float32)
      o_ref[...] = jnp.maximum(z2 + b2_ref[...], 0.0)

  return pl.pallas_call(
      body,
      grid=(nb,),
      in_specs=[
          pl.BlockSpec((rb, hd), lambda i: (i, 0)),
          pl.BlockSpec((rb, 1), lambda i: (i, 0)),
          pl.BlockSpec((g, gfd), lambda i: (0, 0)),
          pl.BlockSpec((hd, hd), lambda i: (0, 0)),
          pl.BlockSpec((hd, hd), lambda i: (0, 0)),
          pl.BlockSpec((gfd, hd), lambda i: (0, 0)),
          pl.BlockSpec((1, hd), lambda i: (0, 0)),
          pl.BlockSpec((hd, hd), lambda i: (0, 0)),
          pl.BlockSpec((1, hd), lambda i: (0, 0)),
      ],
      out_specs=pl.BlockSpec((g, hd), lambda i: (0, 0)),
      out_shape=jax.ShapeDtypeStruct((g, hd), jnp.float32),
      scratch_shapes=[
          pltpu.VMEM((g, hd), jnp.float32),
          pltpu.VMEM((g, hd), jnp.float32),
          pltpu.VMEM((g, hd), jnp.float32),
      ],
  )(h, batch_col, gf, w1a, w1b, w1c, b1.reshape(1, hd), w2, b2.reshape(1, hd))


def kernel(x, edge_index, batch, graph_feat, W_in, b_in,
           Wl0, bl0, Wr0, Wl1, bl1, Wr1, Wl2, bl2, Wr2,
           W1, b1, W2, b2):
  n = x.shape[0]
  hd = W_in.shape[1]
  e = edge_index.shape[1]
  src = edge_index[0]
  dst = edge_index[1]

  z64 = jnp.zeros((n, hd), jnp.float32)
  z16 = jnp.zeros((n, CW), jnp.float32)
  ones16 = jnp.ones((80, CW), jnp.float32)

  seg = _make_sc_segsum(n, hd, e)

  h = _input_proj(x, W_in, b_in)
  for (wl, bl, wr) in ((Wl0, bl0, Wr0), (Wl1, bl1, Wr1), (Wl2, bl2, Wr2)):
    agg2, cnt2 = seg(h, src, dst, z64, z16, ones16)
    h = _layer_combine(agg2, cnt2, h, wl, bl, wr)

  return _pool_mlp(h, batch.reshape(n, 1), graph_feat, W1, b1, W2, b2)


# speedup vs baseline: 9.4142x; 2.0588x over previous
import functools

import jax
import jax.numpy as jnp
from jax import lax
from jax.experimental import pallas as pl
from jax.experimental.pallas import tpu as pltpu
from jax.experimental.pallas import tpu_sc as plsc

NC = 2
NS = 16
CW = 16


@functools.lru_cache(maxsize=None)
def _make_sc_segsum(n_nodes: int, h_dim: int, n_edges: int):
  NW = NC * NS
  epw = n_edges // NW
  K = 80
  nch = epw // K
  GR = 5
  ngr = nch // GR
  assert epw % K == 0 and nch % GR == 0
  rps = (n_nodes // NS) // 8 * 8
  tail = n_nodes - NS * rps

  mesh = plsc.VectorSubcoreMesh(core_axis_name="c", subcore_axis_name="s")

  @functools.partial(
      pl.kernel,
      out_type=(
          jax.ShapeDtypeStruct((NC, n_nodes, h_dim), jnp.float32),
          jax.ShapeDtypeStruct((NC, n_nodes, CW), jnp.float32),
      ),
      mesh=mesh,
      compiler_params=pltpu.CompilerParams(use_tc_tiling_on_sc=False),
      scratch_types=[
          pltpu.VMEM_SHARED((n_nodes, h_dim), jnp.float32),
          pltpu.VMEM_SHARED((n_nodes, CW), jnp.float32),
          pltpu.VMEM((nch, K), jnp.int32),
          pltpu.VMEM((nch, K), jnp.int32),
          pltpu.VMEM((GR * K, h_dim), jnp.float32),
          pltpu.VMEM((K, CW), jnp.float32),
          pltpu.SemaphoreType.DMA,
          pltpu.SemaphoreType.DMA,
          pltpu.SemaphoreType.DMA,
      ],
  )
  def seg(h_hbm, src_hbm, dst_hbm, z64_hbm, z16_hbm, ones_hbm,
          agg_out, cnt_out, agg_s, cnt_s, src_v, dst_v, rows_v, ones_v,
          gsem, ssem, csem):
    c = lax.axis_index("c")
    s = lax.axis_index("s")
    r0 = s * rps
    pltpu.sync_copy(z64_hbm.at[pl.ds(r0, rps)], agg_s.at[pl.ds(r0, rps)])
    pltpu.sync_copy(z16_hbm.at[pl.ds(r0, rps)], cnt_s.at[pl.ds(r0, rps)])
    if tail:
      @pl.when(s == 0)
      def _tail_init():
        t0 = NS * rps
        pltpu.sync_copy(z64_hbm.at[pl.ds(t0, tail)], agg_s.at[pl.ds(t0, tail)])
        pltpu.sync_copy(z16_hbm.at[pl.ds(t0, tail)], cnt_s.at[pl.ds(t0, tail)])
    pltpu.sync_copy(ones_hbm, ones_v)
    w = c * NS + s
    pltpu.sync_copy(src_hbm.at[w], src_v)
    pltpu.sync_copy(dst_hbm.at[w], dst_v)
    plsc.subcore_barrier()

    def body(g, carry):
      gds = []
      for b in range(GR):
        ib = g * GR + b
        gds.append(pltpu.async_copy(
            h_hbm.at[src_v.at[ib]], rows_v.at[pl.ds(b * K, K)], gsem))
      for d in gds:
        d.wait()
      sds = []
      for b in range(GR):
        ib = g * GR + b
        sds.append(pltpu.async_copy(
            rows_v.at[pl.ds(b * K, K)], agg_s.at[dst_v.at[ib]], ssem,
            add=True))
        sds.append(pltpu.async_copy(
            ones_v, cnt_s.at[dst_v.at[ib]], csem, add=True))
      for d in sds:
        d.wait()
      return carry

    lax.fori_loop(0, ngr, body, 0)
    plsc.subcore_barrier()
    pltpu.sync_copy(agg_s.at[pl.ds(r0, rps)], agg_out.at[c, pl.ds(r0, rps)])
    pltpu.sync_copy(cnt_s.at[pl.ds(r0, rps)], cnt_out.at[c, pl.ds(r0, rps)])
    if tail:
      @pl.when(s == 0)
      def _tail_out():
        t0 = NS * rps
        pltpu.sync_copy(agg_s.at[pl.ds(t0, tail)],
                        agg_out.at[c, pl.ds(t0, tail)])
        pltpu.sync_copy(cnt_s.at[pl.ds(t0, tail)],
                        cnt_out.at[c, pl.ds(t0, tail)])

  return seg


def _input_proj(x, w, b):
  n, d = x.shape
  h = w.shape[1]
  rb = 2000

  def body(x_ref, w_ref, b_ref, o_ref):
    acc = jnp.dot(x_ref[...], w_ref[...], preferred_element_type=jnp.float32)
    o_ref[...] = jnp.maximum(acc + b_ref[...], 0.0)

  return pl.pallas_call(
      body,
      grid=(n // rb,),
      in_specs=[
          pl.BlockSpec((rb, d), lambda i: (i, 0)),
          pl.BlockSpec((d, h), lambda i: (0, 0)),
          pl.BlockSpec((1, h), lambda i: (0, 0)),
      ],
      out_specs=pl.BlockSpec((rb, h), lambda i: (i, 0)),
      out_shape=jax.ShapeDtypeStruct((n, h), jnp.float32),
  )(x, w, b.reshape(1, h))


def _layer_combine(agg2, cnt2, h, wl, bl, wr):
  n, hd = h.shape
  rb = 2000

  def body(a_ref, c_ref, h_ref, wl_ref, bl_ref, wr_ref, o_ref):
    a = a_ref[0] + a_ref[1]
    cg = c_ref[0, :, :1] + c_ref[1, :, :1]
    mean = a * (1.0 / jnp.maximum(cg, 1.0))
    acc = jnp.dot(mean, wl_ref[...], preferred_element_type=jnp.float32)
    acc = acc + jnp.dot(h_ref[...], wr_ref[...],
                        preferred_element_type=jnp.float32)
    o_ref[...] = jnp.maximum(acc + bl_ref[...], 0.0)

  return pl.pallas_call(
      body,
      grid=(n // rb,),
      in_specs=[
          pl.BlockSpec((NC, rb, hd), lambda i: (0, i, 0)),
          pl.BlockSpec((NC, rb, CW), lambda i: (0, i, 0)),
          pl.BlockSpec((rb, hd), lambda i: (i, 0)),
          pl.BlockSpec((hd, hd), lambda i: (0, 0)),
          pl.BlockSpec((1, hd), lambda i: (0, 0)),
          pl.BlockSpec((hd, hd), lambda i: (0, 0)),
      ],
      out_specs=pl.BlockSpec((rb, hd), lambda i: (i, 0)),
      out_shape=jax.ShapeDtypeStruct((n, hd), jnp.float32),
  )(agg2, cnt2, h, wl, bl.reshape(1, hd), wr)


def _pool_mlp(h, batch_col, gf, w1, b1, w2, b2):
  n, hd = h.shape
  g = gf.shape[0]
  gfd = gf.shape[1]
  rb = 1000
  nb = n // rb
  w1a = w1[:hd]
  w1b = w1[hd:2 * hd]
  w1c = w1[2 * hd:]

  def body(h_ref, b_ref, gf_ref, w1a_ref, w1b_ref, w1c_ref, b1_ref,
           w2_ref, b2_ref, o_ref, sums, cnts, maxs):
    i = pl.program_id(0)

    @pl.when(i == 0)
    def _init():
      sums[...] = jnp.zeros_like(sums)
      cnts[...] = jnp.zeros_like(cnts)
      maxs[...] = jnp.zeros_like(maxs)

    hb = h_ref[...]
    gid = lax.broadcasted_iota(jnp.int32, (rb, g), 1)
    mask = (b_ref[...] == gid).astype(jnp.float32)
    dn = (((0,), (0,)), ((), ()))
    sums[...] += lax.dot_general(mask, hb, dn,
                                 preferred_element_type=jnp.float32)
    cnts[...] += lax.dot_general(mask, jnp.ones_like(hb), dn,
                                 preferred_element_type=jnp.float32)
    for gc in range(0, g, 8):
      sel = mask[:, gc:gc + 8]
      cand = jnp.max(hb[:, None, :] * sel[:, :, None], axis=0)
      maxs[gc:gc + 8, :] = jnp.maximum(maxs[gc:gc + 8, :], cand)

    @pl.when(i == nb - 1)
    def _final():
      meanp = sums[...] / jnp.maximum(cnts[...], 1.0)
      z1 = jnp.dot(meanp, w1a_ref[...], preferred_element_type=jnp.float32)
      z1 = z1 + jnp.dot(maxs[...], w1b_ref[...],
                        preferred_element_type=jnp.float32)
      z1 = z1 + jnp.dot(gf_ref[...], w1c_ref[...],
                        preferred_element_type=jnp.float32)
      z1 = jnp.maximum(z1 + b1_ref[...], 0.0)
      z2 = jnp.dot(z1, w2_ref[...], preferred_element_type=jnp.float32)
      o_ref[...] = jnp.maximum(z2 + b2_ref[...], 0.0)

  return pl.pallas_call(
      body,
      grid=(nb,),
      in_specs=[
          pl.BlockSpec((rb, hd), lambda i: (i, 0)),
          pl.BlockSpec((rb, 1), lambda i: (i, 0)),
          pl.BlockSpec((g, gfd), lambda i: (0, 0)),
          pl.BlockSpec((hd, hd), lambda i: (0, 0)),
          pl.BlockSpec((hd, hd), lambda i: (0, 0)),
          pl.BlockSpec((gfd, hd), lambda i: (0, 0)),
          pl.BlockSpec((1, hd), lambda i: (0, 0)),
          pl.BlockSpec((hd, hd), lambda i: (0, 0)),
          pl.BlockSpec((1, hd), lambda i: (0, 0)),
      ],
      out_specs=pl.BlockSpec((g, hd), lambda i: (0, 0)),
      out_shape=jax.ShapeDtypeStruct((g, hd), jnp.float32),
      scratch_shapes=[
          pltpu.VMEM((g, hd), jnp.float32),
          pltpu.VMEM((g, hd), jnp.float32),
          pltpu.VMEM((g, hd), jnp.float32),
      ],
  )(h, batch_col, gf, w1a, w1b, w1c, b1.reshape(1, hd), w2, b2.reshape(1, hd))


def kernel(x, edge_index, batch, graph_feat, W_in, b_in,
           Wl0, bl0, Wr0, Wl1, bl1, Wr1, Wl2, bl2, Wr2,
           W1, b1, W2, b2):
  n = x.shape[0]
  hd = W_in.shape[1]
  e = edge_index.shape[1]
  nw = NC * NS
  src = edge_index[0].reshape(nw, -1, 80)
  dst = edge_index[1].reshape(nw, -1, 80)

  z64 = jnp.zeros((n, hd), jnp.float32)
  z16 = jnp.zeros((n, CW), jnp.float32)
  ones16 = jnp.ones((80, CW), jnp.float32)

  seg = _make_sc_segsum(n, hd, e)

  h = _input_proj(x, W_in, b_in)
  for (wl, bl, wr) in ((Wl0, bl0, Wr0), (Wl1, bl1, Wr1), (Wl2, bl2, Wr2)):
    agg2, cnt2 = seg(h, src, dst, z64, z16, ones16)
    h = _layer_combine(agg2, cnt2, h, wl, bl, wr)

  return _pool_mlp(h, batch.reshape(n, 1), graph_feat, W1, b1, W2, b2)


# speedup vs baseline: 10.9015x; 1.1580x over previous
import functools

import jax
import jax.numpy as jnp
from jax import lax
from jax.experimental import pallas as pl
from jax.experimental.pallas import tpu as pltpu
from jax.experimental.pallas import tpu_sc as plsc

NC = 2
NS = 16
CW = 16


@functools.lru_cache(maxsize=None)
def _make_sc_segsum(n_nodes: int, h_dim: int, n_edges: int):
  NW = NC * NS
  epw = n_edges // NW
  K = 80
  nch = epw // K
  GR = 5
  ngr = nch // GR
  assert epw % K == 0 and nch % GR == 0
  rps = (n_nodes // NS) // 8 * 8
  tail = n_nodes - NS * rps

  mesh = plsc.VectorSubcoreMesh(core_axis_name="c", subcore_axis_name="s")

  @functools.partial(
      pl.kernel,
      out_type=(
          jax.ShapeDtypeStruct((NC, n_nodes, h_dim), jnp.float32),
          jax.ShapeDtypeStruct((NC, n_nodes, CW), jnp.float32),
      ),
      mesh=mesh,
      compiler_params=pltpu.CompilerParams(use_tc_tiling_on_sc=False),
      scratch_types=[
          pltpu.VMEM_SHARED((n_nodes, h_dim), jnp.float32),
          pltpu.VMEM_SHARED((n_nodes, CW), jnp.float32),
          pltpu.VMEM((nch, K), jnp.int32),
          pltpu.VMEM((nch, K), jnp.int32),
          pltpu.VMEM((GR * K, h_dim), jnp.float32),
          pltpu.VMEM((K, CW), jnp.float32),
          pltpu.SemaphoreType.DMA,
          pltpu.SemaphoreType.DMA,
          pltpu.SemaphoreType.DMA,
      ],
  )
  def seg(h_hbm, src_hbm, dst_hbm, z64_hbm, z16_hbm, ones_hbm,
          agg_out, cnt_out, agg_s, cnt_s, src_v, dst_v, rows_v, ones_v,
          gsem, ssem, csem):
    c = lax.axis_index("c")
    s = lax.axis_index("s")
    r0 = s * rps
    pltpu.sync_copy(z64_hbm.at[pl.ds(r0, rps)], agg_s.at[pl.ds(r0, rps)])
    pltpu.sync_copy(z16_hbm.at[pl.ds(r0, rps)], cnt_s.at[pl.ds(r0, rps)])
    if tail:
      @pl.when(s == 0)
      def _tail_init():
        t0 = NS * rps
        pltpu.sync_copy(z64_hbm.at[pl.ds(t0, tail)], agg_s.at[pl.ds(t0, tail)])
        pltpu.sync_copy(z16_hbm.at[pl.ds(t0, tail)], cnt_s.at[pl.ds(t0, tail)])
    pltpu.sync_copy(ones_hbm, ones_v)
    w = c * NS + s
    pltpu.sync_copy(src_hbm.at[w], src_v)
    pltpu.sync_copy(dst_hbm.at[w], dst_v)
    plsc.subcore_barrier()

    def body(g, carry):
      gds = []
      for b in range(GR):
        ib = g * GR + b
        gds.append(pltpu.async_copy(
            h_hbm.at[src_v.at[ib]], rows_v.at[pl.ds(b * K, K)], gsem))
      for d in gds:
        d.wait()
      sds = []
      for b in range(GR):
        ib = g * GR + b
        sds.append(pltpu.async_copy(
            rows_v.at[pl.ds(b * K, K)], agg_s.at[dst_v.at[ib]], ssem,
            add=True))
        sds.append(pltpu.async_copy(
            ones_v, cnt_s.at[dst_v.at[ib]], csem, add=True))
      for d in sds:
        d.wait()
      return carry

    lax.fori_loop(0, ngr, body, 0)
    plsc.subcore_barrier()
    pltpu.sync_copy(agg_s.at[pl.ds(r0, rps)], agg_out.at[c, pl.ds(r0, rps)])
    pltpu.sync_copy(cnt_s.at[pl.ds(r0, rps)], cnt_out.at[c, pl.ds(r0, rps)])
    if tail:
      @pl.when(s == 0)
      def _tail_out():
        t0 = NS * rps
        pltpu.sync_copy(agg_s.at[pl.ds(t0, tail)],
                        agg_out.at[c, pl.ds(t0, tail)])
        pltpu.sync_copy(cnt_s.at[pl.ds(t0, tail)],
                        cnt_out.at[c, pl.ds(t0, tail)])

  return seg


def _input_proj(x, w, b):
  n, d = x.shape
  h = w.shape[1]
  rb = 2000

  def body(x_ref, w_ref, b_ref, o_ref):
    acc = jnp.dot(x_ref[...], w_ref[...], preferred_element_type=jnp.float32)
    o_ref[...] = jnp.maximum(acc + b_ref[...], 0.0)

  return pl.pallas_call(
      body,
      grid=(n // rb,),
      in_specs=[
          pl.BlockSpec((rb, d), lambda i: (i, 0)),
          pl.BlockSpec((d, h), lambda i: (0, 0)),
          pl.BlockSpec((1, h), lambda i: (0, 0)),
      ],
      out_specs=pl.BlockSpec((rb, h), lambda i: (i, 0)),
      out_shape=jax.ShapeDtypeStruct((n, h), jnp.float32),
  )(x, w, b.reshape(1, h))


def _layer_combine(agg2, cnt2, h, wl, bl, wr):
  n, hd = h.shape
  rb = 2000

  def body(a_ref, c_ref, h_ref, wl_ref, bl_ref, wr_ref, o_ref):
    a = a_ref[0] + a_ref[1]
    cg = c_ref[0, :, :1] + c_ref[1, :, :1]
    mean = a * (1.0 / jnp.maximum(cg, 1.0))
    acc = jnp.dot(mean, wl_ref[...], preferred_element_type=jnp.float32)
    acc = acc + jnp.dot(h_ref[...], wr_ref[...],
                        preferred_element_type=jnp.float32)
    o_ref[...] = jnp.maximum(acc + bl_ref[...], 0.0)

  return pl.pallas_call(
      body,
      grid=(n // rb,),
      in_specs=[
          pl.BlockSpec((NC, rb, hd), lambda i: (0, i, 0)),
          pl.BlockSpec((NC, rb, CW), lambda i: (0, i, 0)),
          pl.BlockSpec((rb, hd), lambda i: (i, 0)),
          pl.BlockSpec((hd, hd), lambda i: (0, 0)),
          pl.BlockSpec((1, hd), lambda i: (0, 0)),
          pl.BlockSpec((hd, hd), lambda i: (0, 0)),
      ],
      out_specs=pl.BlockSpec((rb, hd), lambda i: (i, 0)),
      out_shape=jax.ShapeDtypeStruct((n, hd), jnp.float32),
  )(agg2, cnt2, h, wl, bl.reshape(1, hd), wr)


def _pool_mlp(h, batch_col, gf, w1, b1, w2, b2):
  n, hd = h.shape
  g = gf.shape[0]
  gfd = gf.shape[1]
  rb = 2000
  nb = n // rb
  w1a = w1[:hd]
  w1b = w1[hd:2 * hd]
  w1c = w1[2 * hd:]

  def body(h_ref, b_ref, gf_ref, w1a_ref, w1b_ref, w1c_ref, b1_ref,
           w2_ref, b2_ref, o_ref, sums, cnts, maxs):
    i = pl.program_id(0)

    @pl.when(i == 0)
    def _init():
      sums[...] = jnp.zeros_like(sums)
      cnts[...] = jnp.zeros_like(cnts)
      maxs[...] = jnp.zeros_like(maxs)

    hb = h_ref[...]
    bfull = jnp.broadcast_to(b_ref[...], (rb, hd))
    gid = lax.broadcasted_iota(jnp.int32, (rb, g), 1)
    mask = (b_ref[...] == gid).astype(jnp.float32)
    dn = (((0,), (0,)), ((), ()))
    sums[...] += lax.dot_general(mask, hb, dn,
                                 preferred_element_type=jnp.float32)
    cnts[...] += lax.dot_general(mask, jnp.ones_like(hb), dn,
                                 preferred_element_type=jnp.float32)
    bmin = b_ref[0, 0]
    bmax = b_ref[rb - 1, 0]

    def gbody(gg, carry):
      @pl.when((gg >= bmin) & (gg <= bmax))
      def _upd():
        m = jnp.max(jnp.where(bfull == gg, hb, 0.0), axis=0, keepdims=True)
        maxs[pl.ds(gg, 1), :] = jnp.maximum(maxs[pl.ds(gg, 1), :], m)
      return carry

    lax.fori_loop(0, g, gbody, 0)

    @pl.when(i == nb - 1)
    def _final():
      meanp = sums[...] / jnp.maximum(cnts[...], 1.0)
      z1 = jnp.dot(meanp, w1a_ref[...], preferred_element_type=jnp.float32)
      z1 = z1 + jnp.dot(maxs[...], w1b_ref[...],
                        preferred_element_type=jnp.float32)
      z1 = z1 + jnp.dot(gf_ref[...], w1c_ref[...],
                        preferred_element_type=jnp.float32)
      z1 = jnp.maximum(z1 + b1_ref[...], 0.0)
      z2 = jnp.dot(z1, w2_ref[...], preferred_element_type=jnp.float32)
      o_ref[...] = jnp.maximum(z2 + b2_ref[...], 0.0)

  return pl.pallas_call(
      body,
      grid=(nb,),
      in_specs=[
          pl.BlockSpec((rb, hd), lambda i: (i, 0)),
          pl.BlockSpec((rb, 1), lambda i: (i, 0)),
          pl.BlockSpec((g, gfd), lambda i: (0, 0)),
          pl.BlockSpec((hd, hd), lambda i: (0, 0)),
          pl.BlockSpec((hd, hd), lambda i: (0, 0)),
          pl.BlockSpec((gfd, hd), lambda i: (0, 0)),
          pl.BlockSpec((1, hd), lambda i: (0, 0)),
          pl.BlockSpec((hd, hd), lambda i: (0, 0)),
          pl.BlockSpec((1, hd), lambda i: (0, 0)),
      ],
      out_specs=pl.BlockSpec((g, hd), lambda i: (0, 0)),
      out_shape=jax.ShapeDtypeStruct((g, hd), jnp.float32),
      scratch_shapes=[
          pltpu.VMEM((g, hd), jnp.float32),
          pltpu.VMEM((g, hd), jnp.float32),
          pltpu.VMEM((g, hd), jnp.float32),
      ],
  )(h, batch_col, gf, w1a, w1b, w1c, b1.reshape(1, hd), w2, b2.reshape(1, hd))


def kernel(x, edge_index, batch, graph_feat, W_in, b_in,
           Wl0, bl0, Wr0, Wl1, bl1, Wr1, Wl2, bl2, Wr2,
           W1, b1, W2, b2):
  n = x.shape[0]
  hd = W_in.shape[1]
  e = edge_index.shape[1]
  nw = NC * NS
  src = edge_index[0].reshape(nw, -1, 80)
  dst = edge_index[1].reshape(nw, -1, 80)

  z64 = jnp.zeros((n, hd), jnp.float32)
  z16 = jnp.zeros((n, CW), jnp.float32)
  ones16 = jnp.ones((80, CW), jnp.float32)

  seg = _make_sc_segsum(n, hd, e)

  h = _input_proj(x, W_in, b_in)
  for (wl, bl, wr) in ((Wl0, bl0, Wr0), (Wl1, bl1, Wr1), (Wl2, bl2, Wr2)):
    agg2, cnt2 = seg(h, src, dst, z64, z16, ones16)
    h = _layer_combine(agg2, cnt2, h, wl, bl, wr)

  return _pool_mlp(h, batch.reshape(n, 1), graph_feat, W1, b1, W2, b2)


# speedup vs baseline: 12.4107x; 1.1384x over previous
import functools

import jax
import jax.numpy as jnp
from jax import lax
from jax.experimental import pallas as pl
from jax.experimental.pallas import tpu as pltpu
from jax.experimental.pallas import tpu_sc as plsc

NC = 2
NS = 16
CW = 16


@functools.lru_cache(maxsize=None)
def _make_sc_segsum(n_nodes: int, h_dim: int, n_edges: int,
                    with_cnt: bool = True):
  NW = NC * NS
  epw = n_edges // NW
  K = 80
  nch = epw // K
  GR = 5
  ngr = nch // GR
  assert epw % K == 0 and nch % GR == 0
  rps = (n_nodes // NS) // 8 * 8
  tail = n_nodes - NS * rps

  mesh = plsc.VectorSubcoreMesh(core_axis_name="c", subcore_axis_name="s")
  params = pltpu.CompilerParams(use_tc_tiling_on_sc=False)

  agg_ty = jax.ShapeDtypeStruct((NC, n_nodes, h_dim), jnp.float32)
  cnt_ty = jax.ShapeDtypeStruct((NC, n_nodes, CW), jnp.float32)
  nbuf = 1 if with_cnt else 2

  scratch = [
      pltpu.VMEM_SHARED((n_nodes, h_dim), jnp.float32),
      pltpu.VMEM((nch, K), jnp.int32),
      pltpu.VMEM((nch, K), jnp.int32),
      pltpu.VMEM((nbuf * GR * K, h_dim), jnp.float32),
      pltpu.SemaphoreType.DMA,
      pltpu.SemaphoreType.DMA,
  ]
  if with_cnt:
    scratch += [
        pltpu.VMEM_SHARED((n_nodes, CW), jnp.float32),
        pltpu.VMEM((K, CW), jnp.float32),
        pltpu.SemaphoreType.DMA,
    ]

  @functools.partial(
      pl.kernel,
      out_type=(agg_ty, cnt_ty) if with_cnt else agg_ty,
      mesh=mesh,
      compiler_params=params,
      scratch_types=scratch,
  )
  def seg(*refs):
    if with_cnt:
      (h_hbm, src_hbm, dst_hbm, z64_hbm, z16_hbm, ones_hbm,
       agg_out, cnt_out, agg_s, src_v, dst_v, rows_v, gsem, ssem,
       cnt_s, ones_v, csem) = refs
    else:
      (h_hbm, src_hbm, dst_hbm, z64_hbm,
       agg_out, agg_s, src_v, dst_v, rows_v, gsem, ssem) = refs
    c = lax.axis_index("c")
    s = lax.axis_index("s")
    r0 = s * rps
    pltpu.sync_copy(z64_hbm.at[pl.ds(r0, rps)], agg_s.at[pl.ds(r0, rps)])
    if with_cnt:
      pltpu.sync_copy(z16_hbm.at[pl.ds(r0, rps)], cnt_s.at[pl.ds(r0, rps)])
      pltpu.sync_copy(ones_hbm, ones_v)
    if tail:
      @pl.when(s == 0)
      def _tail_init():
        t0 = NS * rps
        pltpu.sync_copy(z64_hbm.at[pl.ds(t0, tail)], agg_s.at[pl.ds(t0, tail)])
        if with_cnt:
          pltpu.sync_copy(z16_hbm.at[pl.ds(t0, tail)],
                          cnt_s.at[pl.ds(t0, tail)])
    w = c * NS + s
    pltpu.sync_copy(src_hbm.at[w], src_v)
    pltpu.sync_copy(dst_hbm.at[w], dst_v)
    plsc.subcore_barrier()

    def fire_g(base_ch, half):
      return [pltpu.async_copy(
          h_hbm.at[src_v.at[base_ch + b]],
          rows_v.at[pl.ds((half * GR + b) * K, K)], gsem)
          for b in range(GR)]

    def fire_s(base_ch, half):
      ds = [pltpu.async_copy(
          rows_v.at[pl.ds((half * GR + b) * K, K)],
          agg_s.at[dst_v.at[base_ch + b]], ssem, add=True)
          for b in range(GR)]
      if with_cnt:
        ds += [pltpu.async_copy(ones_v, cnt_s.at[dst_v.at[base_ch + b]],
                                csem, add=True)
               for b in range(GR)]
      return ds

    if with_cnt:
      def body(g, carry):
        base = g * GR
        for d in fire_g(base, 0):
          d.wait()
        for d in fire_s(base, 0):
          d.wait()
        return carry

      lax.fori_loop(0, ngr, body, 0)
    else:
      def body2(g2, carry):
        ca = g2 * 2 * GR
        cb = ca + GR
        ga = fire_g(ca, 0)
        for d in ga:
          d.wait()
        sa = fire_s(ca, 0)
        gb = fire_g(cb, 1)
        for d in gb:
          d.wait()
        sb = fire_s(cb, 1)
        for d in sa:
          d.wait()
        for d in sb:
          d.wait()
        return carry

      lax.fori_loop(0, ngr // 2, body2, 0)
      if ngr % 2:
        ct = (ngr - 1) * GR
        for d in fire_g(ct, 0):
          d.wait()
        for d in fire_s(ct, 0):
          d.wait()

    plsc.subcore_barrier()
    pltpu.sync_copy(agg_s.at[pl.ds(r0, rps)], agg_out.at[c, pl.ds(r0, rps)])
    if with_cnt:
      pltpu.sync_copy(cnt_s.at[pl.ds(r0, rps)], cnt_out.at[c, pl.ds(r0, rps)])
    if tail:
      @pl.when(s == 0)
      def _tail_out():
        t0 = NS * rps
        pltpu.sync_copy(agg_s.at[pl.ds(t0, tail)],
                        agg_out.at[c, pl.ds(t0, tail)])
        if with_cnt:
          pltpu.sync_copy(cnt_s.at[pl.ds(t0, tail)],
                          cnt_out.at[c, pl.ds(t0, tail)])

  return seg


def _input_proj(x, w, b):
  n, d = x.shape
  h = w.shape[1]
  rb = 2000

  def body(x_ref, w_ref, b_ref, o_ref):
    acc = jnp.dot(x_ref[...], w_ref[...], preferred_element_type=jnp.float32)
    o_ref[...] = jnp.maximum(acc + b_ref[...], 0.0)

  return pl.pallas_call(
      body,
      grid=(n // rb,),
      in_specs=[
          pl.BlockSpec((rb, d), lambda i: (i, 0)),
          pl.BlockSpec((d, h), lambda i: (0, 0)),
          pl.BlockSpec((1, h), lambda i: (0, 0)),
      ],
      out_specs=pl.BlockSpec((rb, h), lambda i: (i, 0)),
      out_shape=jax.ShapeDtypeStruct((n, h), jnp.float32),
  )(x, w, b.reshape(1, h))


def _layer_combine(agg2, cnt2, h, wl, bl, wr):
  n, hd = h.shape
  rb = 2000

  def body(a_ref, c_ref, h_ref, wl_ref, bl_ref, wr_ref, o_ref):
    a = a_ref[0] + a_ref[1]
    cg = c_ref[0, :, :1] + c_ref[1, :, :1]
    mean = a * (1.0 / jnp.maximum(cg, 1.0))
    acc = jnp.dot(mean, wl_ref[...], preferred_element_type=jnp.float32)
    acc = acc + jnp.dot(h_ref[...], wr_ref[...],
                        preferred_element_type=jnp.float32)
    o_ref[...] = jnp.maximum(acc + bl_ref[...], 0.0)

  return pl.pallas_call(
      body,
      grid=(n // rb,),
      in_specs=[
          pl.BlockSpec((NC, rb, hd), lambda i: (0, i, 0)),
          pl.BlockSpec((NC, rb, CW), lambda i: (0, i, 0)),
          pl.BlockSpec((rb, hd), lambda i: (i, 0)),
          pl.BlockSpec((hd, hd), lambda i: (0, 0)),
          pl.BlockSpec((1, hd), lambda i: (0, 0)),
          pl.BlockSpec((hd, hd), lambda i: (0, 0)),
      ],
      out_specs=pl.BlockSpec((rb, hd), lambda i: (i, 0)),
      out_shape=jax.ShapeDtypeStruct((n, hd), jnp.float32),
  )(agg2, cnt2, h, wl, bl.reshape(1, hd), wr)


def _combine_pool(agg2, cnt2, h, wl, bl, wr, batch_col, gf, w1, b1, w2, b2):
  n, hd = h.shape
  g = gf.shape[0]
  gfd = gf.shape[1]
  rb = 2000
  nb = n // rb
  w1a = w1[:hd]
  w1b = w1[hd:2 * hd]
  w1c = w1[2 * hd:]

  def body(a_ref, c_ref, h_ref, wl_ref, bl_ref, wr_ref,
           b_ref, gf_ref, w1a_ref, w1b_ref, w1c_ref, b1_ref,
           w2_ref, b2_ref, o_ref, sums, cnts, maxs):
    i = pl.program_id(0)

    @pl.when(i == 0)
    def _init():
      sums[...] = jnp.zeros_like(sums)
      cnts[...] = jnp.zeros_like(cnts)
      maxs[...] = jnp.zeros_like(maxs)

    a = a_ref[0] + a_ref[1]
    cg = c_ref[0, :, :1] + c_ref[1, :, :1]
    mean = a * (1.0 / jnp.maximum(cg, 1.0))
    acc = jnp.dot(mean, wl_ref[...], preferred_element_type=jnp.float32)
    acc = acc + jnp.dot(h_ref[...], wr_ref[...],
                        preferred_element_type=jnp.float32)
    hb = jnp.maximum(acc + bl_ref[...], 0.0)
    bfull = jnp.broadcast_to(b_ref[...], (rb, hd))
    gid = lax.broadcasted_iota(jnp.int32, (rb, g), 1)
    mask = (b_ref[...] == gid).astype(jnp.float32)
    dn = (((0,), (0,)), ((), ()))
    sums[...] += lax.dot_general(mask, hb, dn,
                                 preferred_element_type=jnp.float32)
    cnts[...] += lax.dot_general(mask, jnp.ones_like(hb), dn,
                                 preferred_element_type=jnp.float32)
    bmin = b_ref[0, 0]
    bmax = b_ref[rb - 1, 0]

    def gbody(gg, carry):
      @pl.when((gg >= bmin) & (gg <= bmax))
      def _upd():
        m = jnp.max(jnp.where(bfull == gg, hb, 0.0), axis=0, keepdims=True)
        maxs[pl.ds(gg, 1), :] = jnp.maximum(maxs[pl.ds(gg, 1), :], m)
      return carry

    lax.fori_loop(0, g, gbody, 0)

    @pl.when(i == nb - 1)
    def _final():
      meanp = sums[...] / jnp.maximum(cnts[...], 1.0)
      z1 = jnp.dot(meanp, w1a_ref[...], preferred_element_type=jnp.float32)
      z1 = z1 + jnp.dot(maxs[...], w1b_ref[...],
                        preferred_element_type=jnp.float32)
      z1 = z1 + jnp.dot(gf_ref[...], w1c_ref[...],
                        preferred_element_type=jnp.float32)
      z1 = jnp.maximum(z1 + b1_ref[...], 0.0)
      z2 = jnp.dot(z1, w2_ref[...], preferred_element_type=jnp.float32)
      o_ref[...] = jnp.maximum(z2 + b2_ref[...], 0.0)

  return pl.pallas_call(
      body,
      grid=(nb,),
      in_specs=[
          pl.BlockSpec((NC, rb, hd), lambda i: (0, i, 0)),
          pl.BlockSpec((NC, rb, CW), lambda i: (0, i, 0)),
          pl.BlockSpec((rb, hd), lambda i: (i, 0)),
          pl.BlockSpec((hd, hd), lambda i: (0, 0)),
          pl.BlockSpec((1, hd), lambda i: (0, 0)),
          pl.BlockSpec((hd, hd), lambda i: (0, 0)),
          pl.BlockSpec((rb, 1), lambda i: (i, 0)),
          pl.BlockSpec((g, gfd), lambda i: (0, 0)),
          pl.BlockSpec((hd, hd), lambda i: (0, 0)),
          pl.BlockSpec((hd, hd), lambda i: (0, 0)),
          pl.BlockSpec((gfd, hd), lambda i: (0, 0)),
          pl.BlockSpec((1, hd), lambda i: (0, 0)),
          pl.BlockSpec((hd, hd), lambda i: (0, 0)),
          pl.BlockSpec((1, hd), lambda i: (0, 0)),
      ],
      out_specs=pl.BlockSpec((g, hd), lambda i: (0, 0)),
      out_shape=jax.ShapeDtypeStruct((g, hd), jnp.float32),
      scratch_shapes=[
          pltpu.VMEM((g, hd), jnp.float32),
          pltpu.VMEM((g, hd), jnp.float32),
          pltpu.VMEM((g, hd), jnp.float32),
      ],
  )(agg2, cnt2, h, wl, bl.reshape(1, hd), wr, batch_col, gf,
    w1a, w1b, w1c, b1.reshape(1, hd), w2, b2.reshape(1, hd))


def kernel(x, edge_index, batch, graph_feat, W_in, b_in,
           Wl0, bl0, Wr0, Wl1, bl1, Wr1, Wl2, bl2, Wr2,
           W1, b1, W2, b2):
  n = x.shape[0]
  hd = W_in.shape[1]
  e = edge_index.shape[1]
  nw = NC * NS
  src = edge_index[0].reshape(nw, -1, 80)
  dst = edge_index[1].reshape(nw, -1, 80)

  z64 = jnp.zeros((n, hd), jnp.float32)
  z16 = jnp.zeros((n, CW), jnp.float32)
  ones16 = jnp.ones((80, CW), jnp.float32)

  seg_cnt = _make_sc_segsum(n, hd, e, True)
  seg_nc = _make_sc_segsum(n, hd, e, False)

  h = _input_proj(x, W_in, b_in)
  agg2, cnt2 = seg_cnt(h, src, dst, z64, z16, ones16)
  h = _layer_combine(agg2, cnt2, h, Wl0, bl0, Wr0)
  agg2 = seg_nc(h, src, dst, z64)
  h = _layer_combine(agg2, cnt2, h, Wl1, bl1, Wr1)
  agg2 = seg_nc(h, src, dst, z64)
  return _combine_pool(agg2, cnt2, h, Wl2, bl2, Wr2,
                       batch.reshape(n, 1), graph_feat, W1, b1, W2, b2)


# speedup vs baseline: 13.8577x; 1.1166x over previous
import functools

import jax
import jax.numpy as jnp
from jax import lax
from jax.experimental import pallas as pl
from jax.experimental.pallas import tpu as pltpu
from jax.experimental.pallas import tpu_sc as plsc

NC = 2
NS = 16
CW = 16


@functools.lru_cache(maxsize=None)
def _make_sc_segsum(n_nodes: int, h_dim: int, n_edges: int,
                    with_cnt: bool = False):
  NW = NC * NS
  epw = n_edges // NW
  K = 80
  nch = epw // K
  GR = 5
  ngr = nch // GR
  assert epw % K == 0 and nch % GR == 0
  rps = (n_nodes // NS) // 8 * 8
  tail = n_nodes - NS * rps

  mesh = plsc.VectorSubcoreMesh(core_axis_name="c", subcore_axis_name="s")
  params = pltpu.CompilerParams(use_tc_tiling_on_sc=False)

  agg_ty = jax.ShapeDtypeStruct((NC, n_nodes, h_dim), jnp.float32)
  cnt_ty = jax.ShapeDtypeStruct((NC, n_nodes, CW), jnp.float32)
  scratch = [
      pltpu.VMEM_SHARED((n_nodes, h_dim), jnp.float32),
      pltpu.VMEM((nch, K), jnp.int32),
      pltpu.VMEM((nch, K), jnp.int32),
      pltpu.VMEM((2 * GR * K, h_dim), jnp.float32),
      pltpu.SemaphoreType.DMA,
      pltpu.SemaphoreType.DMA,
      pltpu.SemaphoreType.DMA,
  ]
  if with_cnt:
    scratch += [
        pltpu.VMEM_SHARED((n_nodes, CW), jnp.float32),
        pltpu.VMEM((K, CW), jnp.float32),
        pltpu.SemaphoreType.DMA,
    ]

  @functools.partial(
      pl.kernel,
      out_type=(agg_ty, cnt_ty) if with_cnt else agg_ty,
      mesh=mesh,
      compiler_params=params,
      scratch_types=scratch,
  )
  def seg(*refs):
    if with_cnt:
      (h_hbm, src_hbm, dst_hbm, z_hbm, z16_hbm, ones_hbm, agg_out, cnt_out,
       agg_s, src_v, dst_v, rows_v, gsem, ssem_a, ssem_b,
       cnt_s, ones_v, csem) = refs
    else:
      (h_hbm, src_hbm, dst_hbm, z_hbm, agg_out,
       agg_s, src_v, dst_v, rows_v, gsem, ssem_a, ssem_b) = refs
    c = lax.axis_index("c")
    s = lax.axis_index("s")
    r0 = s * rps
    pltpu.sync_copy(z_hbm.at[pl.ds(r0, rps)], agg_s.at[pl.ds(r0, rps)])
    if with_cnt:
      pltpu.sync_copy(z16_hbm.at[pl.ds(r0, rps)], cnt_s.at[pl.ds(r0, rps)])
      pltpu.sync_copy(ones_hbm, ones_v)
    if tail:
      @pl.when(s == 0)
      def _tail_init():
        t0 = NS * rps
        pltpu.sync_copy(z_hbm.at[pl.ds(t0, tail)], agg_s.at[pl.ds(t0, tail)])
        if with_cnt:
          pltpu.sync_copy(z16_hbm.at[pl.ds(t0, tail)],
                          cnt_s.at[pl.ds(t0, tail)])
    w = c * NS + s
    pltpu.sync_copy(src_hbm.at[w], src_v)
    pltpu.sync_copy(dst_hbm.at[w], dst_v)
    plsc.subcore_barrier()

    def fire_g(base_ch, half):
      return [pltpu.async_copy(
          h_hbm.at[src_v.at[base_ch + b]],
          rows_v.at[pl.ds((half * GR + b) * K, K)], gsem)
          for b in range(GR)]

    def fire_s(base_ch, half, sem):
      ds = [pltpu.async_copy(
          rows_v.at[pl.ds((half * GR + b) * K, K)],
          agg_s.at[dst_v.at[base_ch + b]], sem, add=True)
          for b in range(GR)]
      if with_cnt:
        ds += [pltpu.async_copy(ones_v, cnt_s.at[dst_v.at[base_ch + b]],
                                csem, add=True)
               for b in range(GR)]
      return ds

    def drain_s(sem):
      pltpu.make_async_copy(
          h_hbm.at[pl.ds(0, GR * K)], rows_v.at[pl.ds(0, GR * K)], sem).wait()

    def drain_c():
      if with_cnt:
        for _ in range(GR):
          pltpu.make_async_copy(z16_hbm.at[pl.ds(0, K)], ones_v, csem).wait()

    def body2(g2, carry):
      ca = g2 * 2 * GR
      cb = ca + GR

      @pl.when(g2 > 0)
      def _da():
        drain_s(ssem_a)
        drain_c()

      ga = fire_g(ca, 0)
      for d in ga:
        d.wait()
      fire_s(ca, 0, ssem_a)

      @pl.when(g2 > 0)
      def _db():
        drain_s(ssem_b)
        drain_c()

      gb = fire_g(cb, 1)
      for d in gb:
        d.wait()
      fire_s(cb, 1, ssem_b)
      return carry

    lax.fori_loop(0, ngr // 2, body2, 0)
    drain_s(ssem_a)
    drain_s(ssem_b)
    drain_c()
    drain_c()
    if ngr % 2:
      ct = (ngr - 1) * GR
      for d in fire_g(ct, 0):
        d.wait()
      for d in fire_s(ct, 0, ssem_a):
        d.wait()

    plsc.subcore_barrier()
    pltpu.sync_copy(agg_s.at[pl.ds(r0, rps)], agg_out.at[c, pl.ds(r0, rps)])
    if with_cnt:
      pltpu.sync_copy(cnt_s.at[pl.ds(r0, rps)], cnt_out.at[c, pl.ds(r0, rps)])
    if tail:
      @pl.when(s == 0)
      def _tail_out():
        t0 = NS * rps
        pltpu.sync_copy(agg_s.at[pl.ds(t0, tail)],
                        agg_out.at[c, pl.ds(t0, tail)])
        if with_cnt:
          pltpu.sync_copy(cnt_s.at[pl.ds(t0, tail)],
                          cnt_out.at[c, pl.ds(t0, tail)])

  return seg


def _input_proj(x, w, b):
  n, d = x.shape
  h = w.shape[1]
  rb = 2000

  def body(x_ref, w_ref, b_ref, o_ref):
    acc = jnp.dot(x_ref[...], w_ref[...], preferred_element_type=jnp.float32)
    o_ref[...] = jnp.maximum(acc + b_ref[...], 0.0)

  return pl.pallas_call(
      body,
      grid=(n // rb,),
      in_specs=[
          pl.BlockSpec((rb, d), lambda i: (i, 0)),
          pl.BlockSpec((d, h), lambda i: (0, 0)),
          pl.BlockSpec((1, h), lambda i: (0, 0)),
      ],
      out_specs=pl.BlockSpec((rb, h), lambda i: (i, 0)),
      out_shape=jax.ShapeDtypeStruct((n, h), jnp.float32),
  )(x, w, b.reshape(1, h))


def _layer_combine(agg2, cnt2, h, wl, bl, wr):
  n = h.shape[0]
  hw = h.shape[1]
  aw = agg2.shape[2]
  hd = wl.shape[0]
  rb = 2000
  aug = aw > hd

  def body(*refs):
    if aug:
      a_ref, h_ref, wl_ref, bl_ref, wr_ref, o_ref = refs
    else:
      a_ref, c_ref, h_ref, wl_ref, bl_ref, wr_ref, o_ref = refs
    a = a_ref[0, :, :hd] + a_ref[1, :, :hd]
    if aug:
      cg = a_ref[0, :, hd:hd + 1] + a_ref[1, :, hd:hd + 1]
    else:
      cg = c_ref[0, :, :1] + c_ref[1, :, :1]
    mean = a * (1.0 / jnp.maximum(cg, 1.0))
    acc = jnp.dot(mean, wl_ref[...], preferred_element_type=jnp.float32)
    acc = acc + jnp.dot(h_ref[:, :hd], wr_ref[...],
                        preferred_element_type=jnp.float32)
    o_ref[...] = jnp.maximum(acc + bl_ref[...], 0.0)

  in_specs = [pl.BlockSpec((NC, rb, aw), lambda i: (0, i, 0))]
  args = [agg2]
  if not aug:
    in_specs.append(pl.BlockSpec((NC, rb, CW), lambda i: (0, i, 0)))
    args.append(cnt2)
  in_specs += [
      pl.BlockSpec((rb, hw), lambda i: (i, 0)),
      pl.BlockSpec((hd, hd), lambda i: (0, 0)),
      pl.BlockSpec((1, hd), lambda i: (0, 0)),
      pl.BlockSpec((hd, hd), lambda i: (0, 0)),
  ]
  args += [h, wl, bl.reshape(1, hd), wr]

  return pl.pallas_call(
      body,
      grid=(n // rb,),
      in_specs=in_specs,
      out_specs=pl.BlockSpec((rb, hd), lambda i: (i, 0)),
      out_shape=jax.ShapeDtypeStruct((n, hd), jnp.float32),
  )(*args)


def _combine_pool(agg2, cnt2, h, wl, bl, wr, batch_col, gf, w1, b1, w2, b2):
  n, hd = h.shape
  g = gf.shape[0]
  gfd = gf.shape[1]
  rb = 2000
  nb = n // rb
  w1a = w1[:hd]
  w1b = w1[hd:2 * hd]
  w1c = w1[2 * hd:]

  def body(a_ref, c_ref, h_ref, wl_ref, bl_ref, wr_ref,
           b_ref, gf_ref, w1a_ref, w1b_ref, w1c_ref, b1_ref,
           w2_ref, b2_ref, o_ref, sums, cnts, maxs):
    i = pl.program_id(0)

    @pl.when(i == 0)
    def _init():
      sums[...] = jnp.zeros_like(sums)
      cnts[...] = jnp.zeros_like(cnts)
      maxs[...] = jnp.zeros_like(maxs)

    a = a_ref[0] + a_ref[1]
    cg = c_ref[0, :, :1] + c_ref[1, :, :1]
    mean = a * (1.0 / jnp.maximum(cg, 1.0))
    acc = jnp.dot(mean, wl_ref[...], preferred_element_type=jnp.float32)
    acc = acc + jnp.dot(h_ref[...], wr_ref[...],
                        preferred_element_type=jnp.float32)
    hb = jnp.maximum(acc + bl_ref[...], 0.0)
    bfull = jnp.broadcast_to(b_ref[...], (rb, hd))
    gid = lax.broadcasted_iota(jnp.int32, (rb, g), 1)
    mask = (b_ref[...] == gid).astype(jnp.float32)
    dn = (((0,), (0,)), ((), ()))
    sums[...] += lax.dot_general(mask, hb, dn,
                                 preferred_element_type=jnp.float32)
    cnts[...] += lax.dot_general(mask, jnp.ones_like(hb), dn,
                                 preferred_element_type=jnp.float32)
    bmin = b_ref[0, 0]
    bmax = b_ref[rb - 1, 0]

    def gbody(gg, carry):
      @pl.when((gg >= bmin) & (gg <= bmax))
      def _upd():
        m = jnp.max(jnp.where(bfull == gg, hb, 0.0), axis=0, keepdims=True)
        maxs[pl.ds(gg, 1), :] = jnp.maximum(maxs[pl.ds(gg, 1), :], m)
      return carry

    lax.fori_loop(0, g, gbody, 0)

    @pl.when(i == nb - 1)
    def _final():
      meanp = sums[...] / jnp.maximum(cnts[...], 1.0)
      z1 = jnp.dot(meanp, w1a_ref[...], preferred_element_type=jnp.float32)
      z1 = z1 + jnp.dot(maxs[...], w1b_ref[...],
                        preferred_element_type=jnp.float32)
      z1 = z1 + jnp.dot(gf_ref[...], w1c_ref[...],
                        preferred_element_type=jnp.float32)
      z1 = jnp.maximum(z1 + b1_ref[...], 0.0)
      z2 = jnp.dot(z1, w2_ref[...], preferred_element_type=jnp.float32)
      o_ref[...] = jnp.maximum(z2 + b2_ref[...], 0.0)

  return pl.pallas_call(
      body,
      grid=(nb,),
      in_specs=[
          pl.BlockSpec((NC, rb, hd), lambda i: (0, i, 0)),
          pl.BlockSpec((NC, rb, CW), lambda i: (0, i, 0)),
          pl.BlockSpec((rb, hd), lambda i: (i, 0)),
          pl.BlockSpec((hd, hd), lambda i: (0, 0)),
          pl.BlockSpec((1, hd), lambda i: (0, 0)),
          pl.BlockSpec((hd, hd), lambda i: (0, 0)),
          pl.BlockSpec((rb, 1), lambda i: (i, 0)),
          pl.BlockSpec((g, gfd), lambda i: (0, 0)),
          pl.BlockSpec((hd, hd), lambda i: (0, 0)),
          pl.BlockSpec((hd, hd), lambda i: (0, 0)),
          pl.BlockSpec((gfd, hd), lambda i: (0, 0)),
          pl.BlockSpec((1, hd), lambda i: (0, 0)),
          pl.BlockSpec((hd, hd), lambda i: (0, 0)),
          pl.BlockSpec((1, hd), lambda i: (0, 0)),
      ],
      out_specs=pl.BlockSpec((g, hd), lambda i: (0, 0)),
      out_shape=jax.ShapeDtypeStruct((g, hd), jnp.float32),
      scratch_shapes=[
          pltpu.VMEM((g, hd), jnp.float32),
          pltpu.VMEM((g, hd), jnp.float32),
          pltpu.VMEM((g, hd), jnp.float32),
      ],
  )(agg2, cnt2, h, wl, bl.reshape(1, hd), wr, batch_col, gf,
    w1a, w1b, w1c, b1.reshape(1, hd), w2, b2.reshape(1, hd))


def kernel(x, edge_index, batch, graph_feat, W_in, b_in,
           Wl0, bl0, Wr0, Wl1, bl1, Wr1, Wl2, bl2, Wr2,
           W1, b1, W2, b2):
  n = x.shape[0]
  hd = W_in.shape[1]
  e = edge_index.shape[1]
  nw = NC * NS
  src = edge_index[0].reshape(nw, -1, 80)
  dst = edge_index[1].reshape(nw, -1, 80)

  z64 = jnp.zeros((n, hd), jnp.float32)
  z16 = jnp.zeros((n, CW), jnp.float32)
  ones16 = jnp.ones((80, CW), jnp.float32)

  seg_cnt = _make_sc_segsum(n, hd, e, True)
  seg_nc = _make_sc_segsum(n, hd, e, False)

  h = _input_proj(x, W_in, b_in)
  agg2, cnt2 = seg_cnt(h, src, dst, z64, z16, ones16)
  h = _layer_combine(agg2, cnt2, h, Wl0, bl0, Wr0)
  agg2 = seg_nc(h, src, dst, z64)
  h = _layer_combine(agg2, cnt2, h, Wl1, bl1, Wr1)
  agg2 = seg_nc(h, src, dst, z64)
  return _combine_pool(agg2, cnt2, h, Wl2, bl2, Wr2,
                       batch.reshape(n, 1), graph_feat, W1, b1, W2, b2)


# speedup vs baseline: 14.4259x; 1.0410x over previous
import functools

import jax
import jax.numpy as jnp
from jax import lax
from jax.experimental import pallas as pl
from jax.experimental.pallas import tpu as pltpu
from jax.experimental.pallas import tpu_sc as plsc

NC = 2
NS = 16
CW = 16


@functools.lru_cache(maxsize=None)
def _make_sc_segsum(n_nodes: int, h_dim: int, n_edges: int,
                    with_cnt: bool = False):
  NW = NC * NS
  epw = n_edges // NW
  K = 80
  nch = epw // K
  GR = 5
  ngr = nch // GR
  assert epw % K == 0 and nch % GR == 0
  rps = (n_nodes // NS) // 8 * 8
  tail = n_nodes - NS * rps

  mesh = plsc.VectorSubcoreMesh(core_axis_name="c", subcore_axis_name="s")
  params = pltpu.CompilerParams(use_tc_tiling_on_sc=False)

  agg_ty = jax.ShapeDtypeStruct((NC, n_nodes, h_dim), jnp.float32)
  cnt_ty = jax.ShapeDtypeStruct((NC, n_nodes, CW), jnp.float32)
  scratch = [
      pltpu.VMEM_SHARED((n_nodes, h_dim), jnp.float32),
      pltpu.VMEM((nch, K), jnp.int32),
      pltpu.VMEM((nch, K), jnp.int32),
      pltpu.VMEM((2 * GR * K, h_dim), jnp.float32),
      pltpu.SemaphoreType.DMA,
      pltpu.SemaphoreType.DMA,
      pltpu.SemaphoreType.DMA,
  ]
  if with_cnt:
    scratch += [
        pltpu.VMEM_SHARED((n_nodes, CW), jnp.float32),
        pltpu.VMEM((K, CW), jnp.float32),
        pltpu.SemaphoreType.DMA,
    ]

  @functools.partial(
      pl.kernel,
      out_type=(agg_ty, cnt_ty) if with_cnt else agg_ty,
      mesh=mesh,
      compiler_params=params,
      scratch_types=scratch,
  )
  def seg(*refs):
    if with_cnt:
      (h_hbm, e_hbm, z_hbm, z16_hbm, ones_hbm, agg_out, cnt_out,
       agg_s, src_v, dst_v, rows_v, gsem, ssem_a, ssem_b,
       cnt_s, ones_v, csem) = refs
    else:
      (h_hbm, e_hbm, z_hbm, agg_out,
       agg_s, src_v, dst_v, rows_v, gsem, ssem_a, ssem_b) = refs
    c = lax.axis_index("c")
    s = lax.axis_index("s")
    r0 = s * rps
    pltpu.sync_copy(z_hbm.at[pl.ds(r0, rps)], agg_s.at[pl.ds(r0, rps)])
    if with_cnt:
      pltpu.sync_copy(z16_hbm.at[pl.ds(r0, rps)], cnt_s.at[pl.ds(r0, rps)])
      pltpu.sync_copy(ones_hbm, ones_v)
    if tail:
      @pl.when(s == 0)
      def _tail_init():
        t0 = NS * rps
        pltpu.sync_copy(z_hbm.at[pl.ds(t0, tail)], agg_s.at[pl.ds(t0, tail)])
        if with_cnt:
          pltpu.sync_copy(z16_hbm.at[pl.ds(t0, tail)],
                          cnt_s.at[pl.ds(t0, tail)])
    w = c * NS + s
    pltpu.sync_copy(e_hbm.at[0, w], src_v)
    pltpu.sync_copy(e_hbm.at[1, w], dst_v)
    plsc.subcore_barrier()

    def fire_g(base_ch, half):
      return [pltpu.async_copy(
          h_hbm.at[src_v.at[base_ch + b]],
          rows_v.at[pl.ds((half * GR + b) * K, K)], gsem)
          for b in range(GR)]

    def fire_s(base_ch, half, sem):
      ds = [pltpu.async_copy(
          rows_v.at[pl.ds((half * GR + b) * K, K)],
          agg_s.at[dst_v.at[base_ch + b]], sem, add=True)
          for b in range(GR)]
      if with_cnt:
        ds += [pltpu.async_copy(ones_v, cnt_s.at[dst_v.at[base_ch + b]],
                                csem, add=True)
               for b in range(GR)]
      return ds

    def drain_s(sem):
      pltpu.make_async_copy(
          h_hbm.at[pl.ds(0, GR * K)], rows_v.at[pl.ds(0, GR * K)], sem).wait()

    def drain_c():
      if with_cnt:
        for _ in range(GR):
          pltpu.make_async_copy(z16_hbm.at[pl.ds(0, K)], ones_v, csem).wait()

    def body2(g2, carry):
      ca = g2 * 2 * GR
      cb = ca + GR

      @pl.when(g2 > 0)
      def _da():
        drain_s(ssem_a)
        drain_c()

      ga = fire_g(ca, 0)
      for d in ga:
        d.wait()
      fire_s(ca, 0, ssem_a)

      @pl.when(g2 > 0)
      def _db():
        drain_s(ssem_b)
        drain_c()

      gb = fire_g(cb, 1)
      for d in gb:
        d.wait()
      fire_s(cb, 1, ssem_b)
      return carry

    lax.fori_loop(0, ngr // 2, body2, 0)
    drain_s(ssem_a)
    drain_s(ssem_b)
    drain_c()
    drain_c()
    if ngr % 2:
      ct = (ngr - 1) * GR
      for d in fire_g(ct, 0):
        d.wait()
      for d in fire_s(ct, 0, ssem_a):
        d.wait()

    plsc.subcore_barrier()
    pltpu.sync_copy(agg_s.at[pl.ds(r0, rps)], agg_out.at[c, pl.ds(r0, rps)])
    if with_cnt:
      pltpu.sync_copy(cnt_s.at[pl.ds(r0, rps)], cnt_out.at[c, pl.ds(r0, rps)])
    if tail:
      @pl.when(s == 0)
      def _tail_out():
        t0 = NS * rps
        pltpu.sync_copy(agg_s.at[pl.ds(t0, tail)],
                        agg_out.at[c, pl.ds(t0, tail)])
        if with_cnt:
          pltpu.sync_copy(cnt_s.at[pl.ds(t0, tail)],
                          cnt_out.at[c, pl.ds(t0, tail)])

  return seg


def _input_proj(x, w, b):
  n, d = x.shape
  h = w.shape[1]
  rb = 2000

  def body(x_ref, w_ref, b_ref, o_ref):
    acc = jnp.dot(x_ref[...], w_ref[...], preferred_element_type=jnp.float32)
    o_ref[...] = jnp.maximum(acc + b_ref[...], 0.0)

  return pl.pallas_call(
      body,
      grid=(n // rb,),
      in_specs=[
          pl.BlockSpec((rb, d), lambda i: (i, 0)),
          pl.BlockSpec((d, h), lambda i: (0, 0)),
          pl.BlockSpec((1, h), lambda i: (0, 0)),
      ],
      out_specs=pl.BlockSpec((rb, h), lambda i: (i, 0)),
      out_shape=jax.ShapeDtypeStruct((n, h), jnp.float32),
  )(x, w, b.reshape(1, h))


def _layer_combine(agg2, cnt2, h, wl, bl, wr):
  n = h.shape[0]
  hw = h.shape[1]
  aw = agg2.shape[2]
  hd = wl.shape[0]
  rb = 2000
  aug = aw > hd

  def body(*refs):
    if aug:
      a_ref, h_ref, wl_ref, bl_ref, wr_ref, o_ref = refs
    else:
      a_ref, c_ref, h_ref, wl_ref, bl_ref, wr_ref, o_ref = refs
    a = a_ref[0, :, :hd] + a_ref[1, :, :hd]
    if aug:
      cg = a_ref[0, :, hd:hd + 1] + a_ref[1, :, hd:hd + 1]
    else:
      cg = c_ref[0, :, :1] + c_ref[1, :, :1]
    mean = a * (1.0 / jnp.maximum(cg, 1.0))
    acc = jnp.dot(mean, wl_ref[...], preferred_element_type=jnp.float32)
    acc = acc + jnp.dot(h_ref[:, :hd], wr_ref[...],
                        preferred_element_type=jnp.float32)
    o_ref[...] = jnp.maximum(acc + bl_ref[...], 0.0)

  in_specs = [pl.BlockSpec((NC, rb, aw), lambda i: (0, i, 0))]
  args = [agg2]
  if not aug:
    in_specs.append(pl.BlockSpec((NC, rb, CW), lambda i: (0, i, 0)))
    args.append(cnt2)
  in_specs += [
      pl.BlockSpec((rb, hw), lambda i: (i, 0)),
      pl.BlockSpec((hd, hd), lambda i: (0, 0)),
      pl.BlockSpec((1, hd), lambda i: (0, 0)),
      pl.BlockSpec((hd, hd), lambda i: (0, 0)),
  ]
  args += [h, wl, bl.reshape(1, hd), wr]

  return pl.pallas_call(
      body,
      grid=(n // rb,),
      in_specs=in_specs,
      out_specs=pl.BlockSpec((rb, hd), lambda i: (i, 0)),
      out_shape=jax.ShapeDtypeStruct((n, hd), jnp.float32),
  )(*args)


def _combine_pool(agg2, cnt2, h, wl, bl, wr, batch_col, gf, w1, b1, w2, b2):
  n, hd = h.shape
  g = gf.shape[0]
  gfd = gf.shape[1]
  rb = 2000
  nb = n // rb
  w1a = w1[:hd]
  w1b = w1[hd:2 * hd]
  w1c = w1[2 * hd:]

  def body(a_ref, c_ref, h_ref, wl_ref, bl_ref, wr_ref,
           b_ref, gf_ref, w1a_ref, w1b_ref, w1c_ref, b1_ref,
           w2_ref, b2_ref, o_ref, sums, cnts, maxs):
    i = pl.program_id(0)

    @pl.when(i == 0)
    def _init():
      sums[...] = jnp.zeros_like(sums)
      cnts[...] = jnp.zeros_like(cnts)
      maxs[...] = jnp.zeros_like(maxs)

    a = a_ref[0] + a_ref[1]
    cg = c_ref[0, :, :1] + c_ref[1, :, :1]
    mean = a * (1.0 / jnp.maximum(cg, 1.0))
    acc = jnp.dot(mean, wl_ref[...], preferred_element_type=jnp.float32)
    acc = acc + jnp.dot(h_ref[...], wr_ref[...],
                        preferred_element_type=jnp.float32)
    hb = jnp.maximum(acc + bl_ref[...], 0.0)
    bfull = jnp.broadcast_to(b_ref[...], (rb, hd))
    gid = lax.broadcasted_iota(jnp.int32, (rb, g), 1)
    mask = (b_ref[...] == gid).astype(jnp.float32)
    dn = (((0,), (0,)), ((), ()))
    sums[...] += lax.dot_general(mask, hb, dn,
                                 preferred_element_type=jnp.float32)
    cnts[...] += lax.dot_general(mask, jnp.ones_like(hb), dn,
                                 preferred_element_type=jnp.float32)
    bmin = b_ref[0, 0]
    bmax = b_ref[rb - 1, 0]

    def gbody(gg, carry):
      m = jnp.max(jnp.where(bfull == gg, hb, 0.0), axis=0, keepdims=True)
      maxs[pl.ds(gg, 1), :] = jnp.maximum(maxs[pl.ds(gg, 1), :], m)
      return carry

    lax.fori_loop(bmin, bmax + 1, gbody, 0)

    @pl.when(i == nb - 1)
    def _final():
      meanp = sums[...] / jnp.maximum(cnts[...], 1.0)
      z1 = jnp.dot(meanp, w1a_ref[...], preferred_element_type=jnp.float32)
      z1 = z1 + jnp.dot(maxs[...], w1b_ref[...],
                        preferred_element_type=jnp.float32)
      z1 = z1 + jnp.dot(gf_ref[...], w1c_ref[...],
                        preferred_element_type=jnp.float32)
      z1 = jnp.maximum(z1 + b1_ref[...], 0.0)
      z2 = jnp.dot(z1, w2_ref[...], preferred_element_type=jnp.float32)
      o_ref[...] = jnp.maximum(z2 + b2_ref[...], 0.0)

  return pl.pallas_call(
      body,
      grid=(nb,),
      in_specs=[
          pl.BlockSpec((NC, rb, hd), lambda i: (0, i, 0)),
          pl.BlockSpec((NC, rb, CW), lambda i: (0, i, 0)),
          pl.BlockSpec((rb, hd), lambda i: (i, 0)),
          pl.BlockSpec((hd, hd), lambda i: (0, 0)),
          pl.BlockSpec((1, hd), lambda i: (0, 0)),
          pl.BlockSpec((hd, hd), lambda i: (0, 0)),
          pl.BlockSpec((rb, 1), lambda i: (i, 0)),
          pl.BlockSpec((g, gfd), lambda i: (0, 0)),
          pl.BlockSpec((hd, hd), lambda i: (0, 0)),
          pl.BlockSpec((hd, hd), lambda i: (0, 0)),
          pl.BlockSpec((gfd, hd), lambda i: (0, 0)),
          pl.BlockSpec((1, hd), lambda i: (0, 0)),
          pl.BlockSpec((hd, hd), lambda i: (0, 0)),
          pl.BlockSpec((1, hd), lambda i: (0, 0)),
      ],
      out_specs=pl.BlockSpec((g, hd), lambda i: (0, 0)),
      out_shape=jax.ShapeDtypeStruct((g, hd), jnp.float32),
      scratch_shapes=[
          pltpu.VMEM((g, hd), jnp.float32),
          pltpu.VMEM((g, hd), jnp.float32),
          pltpu.VMEM((g, hd), jnp.float32),
      ],
  )(agg2, cnt2, h, wl, bl.reshape(1, hd), wr, batch_col, gf,
    w1a, w1b, w1c, b1.reshape(1, hd), w2, b2.reshape(1, hd))


def kernel(x, edge_index, batch, graph_feat, W_in, b_in,
           Wl0, bl0, Wr0, Wl1, bl1, Wr1, Wl2, bl2, Wr2,
           W1, b1, W2, b2):
  n = x.shape[0]
  hd = W_in.shape[1]
  e = edge_index.shape[1]
  nw = NC * NS
  edges = edge_index.reshape(2, nw, -1, 80)

  z64 = jnp.zeros((n, hd), jnp.float32)
  z16 = jnp.zeros((n, CW), jnp.float32)
  ones16 = jnp.ones((80, CW), jnp.float32)

  seg_cnt = _make_sc_segsum(n, hd, e, True)
  seg_nc = _make_sc_segsum(n, hd, e, False)

  h = _input_proj(x, W_in, b_in)
  agg2, cnt2 = seg_cnt(h, edges, z64, z16, ones16)
  h = _layer_combine(agg2, cnt2, h, Wl0, bl0, Wr0)
  agg2 = seg_nc(h, edges, z64)
  h = _layer_combine(agg2, cnt2, h, Wl1, bl1, Wr1)
  agg2 = seg_nc(h, edges, z64)
  return _combine_pool(agg2, cnt2, h, Wl2, bl2, Wr2,
                       batch.reshape(n, 1), graph_feat, W1, b1, W2, b2)


# speedup vs baseline: 15.8390x; 1.0980x over previous
import functools

import jax
import jax.numpy as jnp
from jax import lax
from jax.experimental import pallas as pl
from jax.experimental.pallas import tpu as pltpu
from jax.experimental.pallas import tpu_sc as plsc

NC = 2
NS = 16
CW = 16


@functools.lru_cache(maxsize=None)
def _make_sc_segsum(n_nodes: int, h_dim: int, n_edges: int,
                    with_cnt: bool = False):
  NW = NC * NS
  epw = n_edges // NW
  K = 80
  nch = epw // K
  GR = 5
  ngr = nch // GR
  assert epw % K == 0 and nch % GR == 0
  rps = (n_nodes // NS) // 8 * 8
  tail = n_nodes - NS * rps

  mesh = plsc.VectorSubcoreMesh(core_axis_name="c", subcore_axis_name="s")
  params = pltpu.CompilerParams(use_tc_tiling_on_sc=False)

  OW = 128
  agg_ty = jax.ShapeDtypeStruct((NC, n_nodes, OW), jnp.float32)
  scratch = [
      pltpu.VMEM_SHARED((n_nodes, h_dim), jnp.float32),
      pltpu.VMEM((nch, K), jnp.int32),
      pltpu.VMEM((nch, K), jnp.int32),
      pltpu.VMEM((2 * GR * K, h_dim), jnp.float32),
      pltpu.SemaphoreType.DMA,
      pltpu.SemaphoreType.DMA,
      pltpu.SemaphoreType.DMA,
  ]
  if with_cnt:
    scratch += [
        pltpu.VMEM_SHARED((n_nodes, CW), jnp.float32),
        pltpu.VMEM((K, CW), jnp.float32),
        pltpu.SemaphoreType.DMA,
    ]

  @functools.partial(
      pl.kernel,
      out_type=agg_ty,
      mesh=mesh,
      compiler_params=params,
      scratch_types=scratch,
  )
  def seg(*refs):
    if with_cnt:
      (h_hbm, e_hbm, z_hbm, z16_hbm, ones_hbm, agg_out,
       agg_s, src_v, dst_v, rows_v, gsem, ssem_a, ssem_b,
       cnt_s, ones_v, csem) = refs
    else:
      (h_hbm, e_hbm, z_hbm, agg_out,
       agg_s, src_v, dst_v, rows_v, gsem, ssem_a, ssem_b) = refs
    c = lax.axis_index("c")
    s = lax.axis_index("s")
    r0 = s * rps
    pltpu.sync_copy(z_hbm.at[pl.ds(r0, rps)], agg_s.at[pl.ds(r0, rps)])
    if with_cnt:
      pltpu.sync_copy(z16_hbm.at[pl.ds(r0, rps)], cnt_s.at[pl.ds(r0, rps)])
      pltpu.sync_copy(ones_hbm, ones_v)
    if tail:
      @pl.when(s == 0)
      def _tail_init():
        t0 = NS * rps
        pltpu.sync_copy(z_hbm.at[pl.ds(t0, tail)], agg_s.at[pl.ds(t0, tail)])
        if with_cnt:
          pltpu.sync_copy(z16_hbm.at[pl.ds(t0, tail)],
                          cnt_s.at[pl.ds(t0, tail)])
    w = c * NS + s
    pltpu.sync_copy(e_hbm.at[0, w], src_v)
    pltpu.sync_copy(e_hbm.at[1, w], dst_v)
    plsc.subcore_barrier()

    def fire_g(base_ch, half):
      return [pltpu.async_copy(
          h_hbm.at[src_v.at[base_ch + b]],
          rows_v.at[pl.ds((half * GR + b) * K, K)], gsem)
          for b in range(GR)]

    def fire_s(base_ch, half, sem):
      ds = [pltpu.async_copy(
          rows_v.at[pl.ds((half * GR + b) * K, K)],
          agg_s.at[dst_v.at[base_ch + b]], sem, add=True)
          for b in range(GR)]
      if with_cnt:
        ds += [pltpu.async_copy(ones_v, cnt_s.at[dst_v.at[base_ch + b]],
                                csem, add=True)
               for b in range(GR)]
      return ds

    def drain_s(sem):
      pltpu.make_async_copy(
          h_hbm.at[pl.ds(0, GR * K)], rows_v.at[pl.ds(0, GR * K)], sem).wait()

    def drain_c():
      if with_cnt:
        for _ in range(GR):
          pltpu.make_async_copy(z16_hbm.at[pl.ds(0, K)], ones_v, csem).wait()

    def body2(g2, carry):
      ca = g2 * 2 * GR
      cb = ca + GR

      @pl.when(g2 > 0)
      def _da():
        drain_s(ssem_a)
        drain_c()

      ga = fire_g(ca, 0)
      for d in ga:
        d.wait()
      fire_s(ca, 0, ssem_a)

      @pl.when(g2 > 0)
      def _db():
        drain_s(ssem_b)
        drain_c()

      gb = fire_g(cb, 1)
      for d in gb:
        d.wait()
      fire_s(cb, 1, ssem_b)
      return carry

    lax.fori_loop(0, ngr // 2, body2, 0)
    drain_s(ssem_a)
    drain_s(ssem_b)
    drain_c()
    drain_c()
    if ngr % 2:
      ct = (ngr - 1) * GR
      for d in fire_g(ct, 0):
        d.wait()
      for d in fire_s(ct, 0, ssem_a):
        d.wait()

    plsc.subcore_barrier()

    def wb(rlo, nrows):
      pltpu.sync_copy(agg_s.at[pl.ds(rlo, nrows)],
                      agg_out.at[c, pl.ds(rlo, nrows), pl.ds(0, h_dim)])
      if with_cnt:
        pltpu.sync_copy(cnt_s.at[pl.ds(rlo, nrows)],
                        agg_out.at[c, pl.ds(rlo, nrows), pl.ds(h_dim, CW)])

    wb(r0, rps)
    if tail:
      @pl.when(s == 0)
      def _tail_out():
        wb(NS * rps, tail)

  return seg


def _input_proj(x, w, b):
  n, d = x.shape
  h = w.shape[1]
  rb = 2000

  def body(x_ref, w_ref, b_ref, o_ref):
    acc = jnp.dot(x_ref[...], w_ref[...], preferred_element_type=jnp.float32)
    o_ref[...] = jnp.maximum(acc + b_ref[...], 0.0)

  return pl.pallas_call(
      body,
      grid=(n // rb,),
      in_specs=[
          pl.BlockSpec((rb, d), lambda i: (i, 0)),
          pl.BlockSpec((d, h), lambda i: (0, 0)),
          pl.BlockSpec((1, h), lambda i: (0, 0)),
      ],
      out_specs=pl.BlockSpec((rb, h), lambda i: (i, 0)),
      out_shape=jax.ShapeDtypeStruct((n, h), jnp.float32),
  )(x, w, b.reshape(1, h))


def _layer_combine(agg2, cnt2, h, wl, bl, wr):
  n = h.shape[0]
  hw = h.shape[1]
  aw = agg2.shape[2]
  hd = wl.shape[0]
  rb = 2000
  aug = cnt2 is None

  def body(*refs):
    if aug:
      a_ref, h_ref, wl_ref, bl_ref, wr_ref, o_ref = refs
    else:
      a_ref, c_ref, h_ref, wl_ref, bl_ref, wr_ref, o_ref = refs
    a = a_ref[0, :, :hd] + a_ref[1, :, :hd]
    if aug:
      cg = a_ref[0, :, hd:hd + 1] + a_ref[1, :, hd:hd + 1]
    else:
      cg = c_ref[0, :, :1] + c_ref[1, :, :1]
    mean = a * (1.0 / jnp.maximum(cg, 1.0))
    acc = jnp.dot(mean, wl_ref[...], preferred_element_type=jnp.float32)
    acc = acc + jnp.dot(h_ref[:, :hd], wr_ref[...],
                        preferred_element_type=jnp.float32)
    o_ref[...] = jnp.maximum(acc + bl_ref[...], 0.0)

  in_specs = [pl.BlockSpec((NC, rb, aw), lambda i: (0, i, 0))]
  args = [agg2]
  if not aug:
    in_specs.append(pl.BlockSpec((NC, rb, CW), lambda i: (0, i, 0)))
    args.append(cnt2)
  in_specs += [
      pl.BlockSpec((rb, hw), lambda i: (i, 0)),
      pl.BlockSpec((hd, hd), lambda i: (0, 0)),
      pl.BlockSpec((1, hd), lambda i: (0, 0)),
      pl.BlockSpec((hd, hd), lambda i: (0, 0)),
  ]
  args += [h, wl, bl.reshape(1, hd), wr]

  return pl.pallas_call(
      body,
      grid=(n // rb,),
      in_specs=in_specs,
      out_specs=pl.BlockSpec((rb, hd), lambda i: (i, 0)),
      out_shape=jax.ShapeDtypeStruct((n, hd), jnp.float32),
  )(*args)


def _combine_pool(agg2, cnt2, h, wl, bl, wr, batch_col, gf, w1, b1, w2, b2):
  n, hd = h.shape
  aw = agg2.shape[2]
  g = gf.shape[0]
  gfd = gf.shape[1]
  rb = 400
  nb = n // rb
  w1a = w1[:hd]
  w1b = w1[hd:2 * hd]
  w1c = w1[2 * hd:]

  def body(a_ref, c_ref, h_ref, wl_ref, bl_ref, wr_ref,
           b_ref, gf_ref, w1a_ref, w1b_ref, w1c_ref, b1_ref,
           w2_ref, b2_ref, o_ref, sums, cnts, maxs):
    i = pl.program_id(0)

    @pl.when(i == 0)
    def _init():
      sums[...] = jnp.zeros_like(sums)
      cnts[...] = jnp.zeros_like(cnts)
      maxs[...] = jnp.zeros_like(maxs)

    a = a_ref[0, :, :hd] + a_ref[1, :, :hd]
    cg = c_ref[0, :, :1] + c_ref[1, :, :1]
    mean = a * (1.0 / jnp.maximum(cg, 1.0))
    acc = jnp.dot(mean, wl_ref[...], preferred_element_type=jnp.float32)
    acc = acc + jnp.dot(h_ref[...], wr_ref[...],
                        preferred_element_type=jnp.float32)
    hb = jnp.maximum(acc + bl_ref[...], 0.0)
    bfull = jnp.broadcast_to(b_ref[...], (rb, hd))
    gid = lax.broadcasted_iota(jnp.int32, (rb, g), 1)
    mask = (b_ref[...] == gid).astype(jnp.float32)
    dn = (((0,), (0,)), ((), ()))
    sums[...] += lax.dot_general(mask, hb, dn,
                                 preferred_element_type=jnp.float32)
    cnts[...] += lax.dot_general(mask, jnp.ones_like(hb), dn,
                                 preferred_element_type=jnp.float32)
    bmin = b_ref[0, 0]
    bmax = b_ref[rb - 1, 0]

    def gbody(gg, carry):
      m = jnp.max(jnp.where(bfull == gg, hb, 0.0), axis=0, keepdims=True)
      maxs[pl.ds(gg, 1), :] = jnp.maximum(maxs[pl.ds(gg, 1), :], m)
      return carry

    lax.fori_loop(bmin, bmax + 1, gbody, 0)

    @pl.when(i == nb - 1)
    def _final():
      meanp = sums[...] / jnp.maximum(cnts[...], 1.0)
      z1 = jnp.dot(meanp, w1a_ref[...], preferred_element_type=jnp.float32)
      z1 = z1 + jnp.dot(maxs[...], w1b_ref[...],
                        preferred_element_type=jnp.float32)
      z1 = z1 + jnp.dot(gf_ref[...], w1c_ref[...],
                        preferred_element_type=jnp.float32)
      z1 = jnp.maximum(z1 + b1_ref[...], 0.0)
      z2 = jnp.dot(z1, w2_ref[...], preferred_element_type=jnp.float32)
      o_ref[...] = jnp.maximum(z2 + b2_ref[...], 0.0)

  return pl.pallas_call(
      body,
      grid=(nb,),
      in_specs=[
          pl.BlockSpec((NC, rb, aw), lambda i: (0, i, 0)),
          pl.BlockSpec((NC, rb, CW), lambda i: (0, i, 0)),
          pl.BlockSpec((rb, hd), lambda i: (i, 0)),
          pl.BlockSpec((hd, hd), lambda i: (0, 0)),
          pl.BlockSpec((1, hd), lambda i: (0, 0)),
          pl.BlockSpec((hd, hd), lambda i: (0, 0)),
          pl.BlockSpec((rb, 1), lambda i: (i, 0)),
          pl.BlockSpec((g, gfd), lambda i: (0, 0)),
          pl.BlockSpec((hd, hd), lambda i: (0, 0)),
          pl.BlockSpec((hd, hd), lambda i: (0, 0)),
          pl.BlockSpec((gfd, hd), lambda i: (0, 0)),
          pl.BlockSpec((1, hd), lambda i: (0, 0)),
          pl.BlockSpec((hd, hd), lambda i: (0, 0)),
          pl.BlockSpec((1, hd), lambda i: (0, 0)),
      ],
      out_specs=pl.BlockSpec((g, hd), lambda i: (0, 0)),
      out_shape=jax.ShapeDtypeStruct((g, hd), jnp.float32),
      scratch_shapes=[
          pltpu.VMEM((g, hd), jnp.float32),
          pltpu.VMEM((g, hd), jnp.float32),
          pltpu.VMEM((g, hd), jnp.float32),
      ],
  )(agg2, cnt2, h, wl, bl.reshape(1, hd), wr, batch_col, gf,
    w1a, w1b, w1c, b1.reshape(1, hd), w2, b2.reshape(1, hd))


def kernel(x, edge_index, batch, graph_feat, W_in, b_in,
           Wl0, bl0, Wr0, Wl1, bl1, Wr1, Wl2, bl2, Wr2,
           W1, b1, W2, b2):
  n = x.shape[0]
  hd = W_in.shape[1]
  e = edge_index.shape[1]
  nw = NC * NS
  edges = edge_index.reshape(2, nw, -1, 80)

  z64 = jnp.zeros((n, hd), jnp.float32)
  z16 = jnp.zeros((n, CW), jnp.float32)
  ones16 = jnp.ones((80, CW), jnp.float32)

  seg_cnt = _make_sc_segsum(n, hd, e, True)
  seg_nc = _make_sc_segsum(n, hd, e, False)

  h = _input_proj(x, W_in, b_in)
  agg2 = seg_cnt(h, edges, z64, z16, ones16)
  cnt2 = lax.slice(agg2, (0, 0, hd), (NC, n, hd + CW))
  h = _layer_combine(agg2, None, h, Wl0, bl0, Wr0)
  agg2 = seg_nc(h, edges, z64)
  h = _layer_combine(agg2, cnt2, h, Wl1, bl1, Wr1)
  agg2 = seg_nc(h, edges, z64)
  return _combine_pool(agg2, cnt2, h, Wl2, bl2, Wr2,
                       batch.reshape(n, 1), graph_feat, W1, b1, W2, b2)


# speedup vs baseline: 16.1888x; 1.0221x over previous
import functools

import jax
import jax.numpy as jnp
from jax import lax
from jax.experimental import pallas as pl
from jax.experimental.pallas import tpu as pltpu
from jax.experimental.pallas import tpu_sc as plsc

NC = 2
NS = 16
CW = 16


@functools.lru_cache(maxsize=None)
def _make_sc_segsum(n_nodes: int, h_dim: int, n_edges: int,
                    with_cnt: bool = False):
  NW = NC * NS
  epw = n_edges // NW
  K = 80
  nch = epw // K
  GR = 5
  ngr = nch // GR
  assert epw % K == 0 and nch % GR == 0
  rps = (n_nodes // NS) // 8 * 8
  tail = n_nodes - NS * rps

  mesh = plsc.VectorSubcoreMesh(core_axis_name="c", subcore_axis_name="s")
  params = pltpu.CompilerParams(use_tc_tiling_on_sc=False)

  OW = 128
  agg_ty = jax.ShapeDtypeStruct((NC, n_nodes, OW), jnp.float32)
  scratch = [
      pltpu.VMEM_SHARED((n_nodes, h_dim), jnp.float32),
      pltpu.VMEM((nch, K), jnp.int32),
      pltpu.VMEM((nch, K), jnp.int32),
      pltpu.VMEM((2 * GR * K, h_dim), jnp.float32),
      pltpu.SemaphoreType.DMA,
      pltpu.SemaphoreType.DMA,
      pltpu.SemaphoreType.DMA,
  ]
  if with_cnt:
    scratch += [
        pltpu.VMEM_SHARED((n_nodes, CW), jnp.float32),
        pltpu.VMEM((K, CW), jnp.float32),
        pltpu.SemaphoreType.DMA,
    ]

  @functools.partial(
      pl.kernel,
      out_type=agg_ty,
      mesh=mesh,
      compiler_params=params,
      scratch_types=scratch,
  )
  def seg(*refs):
    if with_cnt:
      (h_hbm, e_hbm, z_hbm, z16_hbm, ones_hbm, agg_out,
       agg_s, src_v, dst_v, rows_v, gsem, ssem_a, ssem_b,
       cnt_s, ones_v, csem) = refs
    else:
      (h_hbm, e_hbm, z_hbm, agg_out,
       agg_s, src_v, dst_v, rows_v, gsem, ssem_a, ssem_b) = refs
    c = lax.axis_index("c")
    s = lax.axis_index("s")
    r0 = s * rps
    pltpu.sync_copy(z_hbm.at[pl.ds(r0, rps)], agg_s.at[pl.ds(r0, rps)])
    if with_cnt:
      pltpu.sync_copy(z16_hbm.at[pl.ds(r0, rps)], cnt_s.at[pl.ds(r0, rps)])
      pltpu.sync_copy(ones_hbm, ones_v)
    if tail:
      @pl.when(s == 0)
      def _tail_init():
        t0 = NS * rps
        pltpu.sync_copy(z_hbm.at[pl.ds(t0, tail)], agg_s.at[pl.ds(t0, tail)])
        if with_cnt:
          pltpu.sync_copy(z16_hbm.at[pl.ds(t0, tail)],
                          cnt_s.at[pl.ds(t0, tail)])
    w = c * NS + s
    pltpu.sync_copy(e_hbm.at[0, w], src_v)
    pltpu.sync_copy(e_hbm.at[1, w], dst_v)
    plsc.subcore_barrier()

    def fire_g(base_ch, half):
      return [pltpu.async_copy(
          h_hbm.at[src_v.at[base_ch + b]],
          rows_v.at[pl.ds((half * GR + b) * K, K)], gsem)
          for b in range(GR)]

    def fire_s(base_ch, half, sem):
      ds = [pltpu.async_copy(
          rows_v.at[pl.ds((half * GR + b) * K, K)],
          agg_s.at[dst_v.at[base_ch + b]], sem, add=True)
          for b in range(GR)]
      if with_cnt:
        ds += [pltpu.async_copy(ones_v, cnt_s.at[dst_v.at[base_ch + b]],
                                csem, add=True)
               for b in range(GR)]
      return ds

    def drain_s(sem):
      pltpu.make_async_copy(
          h_hbm.at[pl.ds(0, GR * K)], rows_v.at[pl.ds(0, GR * K)], sem).wait()

    def drain_c():
      if with_cnt:
        for _ in range(GR):
          pltpu.make_async_copy(z16_hbm.at[pl.ds(0, K)], ones_v, csem).wait()

    def body2(g2, carry):
      ca = g2 * 2 * GR
      cb = ca + GR

      @pl.when(g2 > 0)
      def _da():
        drain_s(ssem_a)
        drain_c()

      ga = fire_g(ca, 0)
      for d in ga:
        d.wait()
      fire_s(ca, 0, ssem_a)

      @pl.when(g2 > 0)
      def _db():
        drain_s(ssem_b)
        drain_c()

      gb = fire_g(cb, 1)
      for d in gb:
        d.wait()
      fire_s(cb, 1, ssem_b)
      return carry

    lax.fori_loop(0, ngr // 2, body2, 0)
    drain_s(ssem_a)
    drain_s(ssem_b)
    drain_c()
    drain_c()
    if ngr % 2:
      ct = (ngr - 1) * GR
      for d in fire_g(ct, 0):
        d.wait()
      for d in fire_s(ct, 0, ssem_a):
        d.wait()

    plsc.subcore_barrier()

    def wb(rlo, nrows):
      pltpu.sync_copy(agg_s.at[pl.ds(rlo, nrows)],
                      agg_out.at[c, pl.ds(rlo, nrows), pl.ds(0, h_dim)])
      if with_cnt:
        pltpu.sync_copy(cnt_s.at[pl.ds(rlo, nrows)],
                        agg_out.at[c, pl.ds(rlo, nrows), pl.ds(h_dim, CW)])

    wb(r0, rps)
    if tail:
      @pl.when(s == 0)
      def _tail_out():
        wb(NS * rps, tail)

  return seg


def _input_proj(x, w, b):
  n, d = x.shape
  h = w.shape[1]
  rb = 2000

  def body(x_ref, w_ref, b_ref, o_ref):
    acc = jnp.dot(x_ref[...], w_ref[...], preferred_element_type=jnp.float32)
    o_ref[...] = jnp.maximum(acc + b_ref[...], 0.0)

  return pl.pallas_call(
      body,
      grid=(n // rb,),
      in_specs=[
          pl.BlockSpec((rb, d), lambda i: (i, 0)),
          pl.BlockSpec((d, h), lambda i: (0, 0)),
          pl.BlockSpec((1, h), lambda i: (0, 0)),
      ],
      out_specs=pl.BlockSpec((rb, h), lambda i: (i, 0)),
      out_shape=jax.ShapeDtypeStruct((n, h), jnp.float32),
  )(x, w, b.reshape(1, h))


def _layer_combine(agg2, cnt2, h, wl, bl, wr):
  n = h.shape[0]
  hw = h.shape[1]
  aw = agg2.shape[2]
  hd = wl.shape[0]
  rb = 2000
  aug = cnt2 is None

  def body(*refs):
    if aug:
      a_ref, h_ref, wl_ref, bl_ref, wr_ref, o_ref = refs
    else:
      a_ref, c_ref, h_ref, wl_ref, bl_ref, wr_ref, o_ref = refs
    a = a_ref[0, :, :hd] + a_ref[1, :, :hd]
    if aug:
      cg = a_ref[0, :, hd:hd + 1] + a_ref[1, :, hd:hd + 1]
    else:
      cg = c_ref[0, :, :1] + c_ref[1, :, :1]
    mean = a * (1.0 / jnp.maximum(cg, 1.0))
    acc = jnp.dot(mean, wl_ref[...], preferred_element_type=jnp.float32)
    acc = acc + jnp.dot(h_ref[:, :hd], wr_ref[...],
                        preferred_element_type=jnp.float32)
    o_ref[...] = jnp.maximum(acc + bl_ref[...], 0.0)

  in_specs = [pl.BlockSpec((NC, rb, aw), lambda i: (0, i, 0))]
  args = [agg2]
  if not aug:
    in_specs.append(pl.BlockSpec((NC, rb, CW), lambda i: (0, i, 0)))
    args.append(cnt2)
  in_specs += [
      pl.BlockSpec((rb, hw), lambda i: (i, 0)),
      pl.BlockSpec((hd, hd), lambda i: (0, 0)),
      pl.BlockSpec((1, hd), lambda i: (0, 0)),
      pl.BlockSpec((hd, hd), lambda i: (0, 0)),
  ]
  args += [h, wl, bl.reshape(1, hd), wr]

  return pl.pallas_call(
      body,
      grid=(n // rb,),
      in_specs=in_specs,
      out_specs=pl.BlockSpec((rb, hd), lambda i: (i, 0)),
      out_shape=jax.ShapeDtypeStruct((n, hd), jnp.float32),
  )(*args)


def _combine_pool(agg2, cnt2, h, wl, bl, wr, batch_col, gf, w1, b1, w2, b2):
  n, hd = h.shape
  aw = agg2.shape[2]
  g = gf.shape[0]
  gfd = gf.shape[1]
  rb = 1000
  nb = n // rb
  w1a = w1[:hd]
  w1b = w1[hd:2 * hd]
  w1c = w1[2 * hd:]

  def body(a_ref, c_ref, h_ref, wl_ref, bl_ref, wr_ref,
           b_ref, gf_ref, w1a_ref, w1b_ref, w1c_ref, b1_ref,
           w2_ref, b2_ref, o_ref, sums, cnts, maxs):
    i = pl.program_id(0)

    @pl.when(i == 0)
    def _init():
      sums[...] = jnp.zeros_like(sums)
      cnts[...] = jnp.zeros_like(cnts)
      maxs[...] = jnp.zeros_like(maxs)

    a = a_ref[0, :, :hd] + a_ref[1, :, :hd]
    cg = c_ref[0, :, :1] + c_ref[1, :, :1]
    mean = a * (1.0 / jnp.maximum(cg, 1.0))
    acc = jnp.dot(mean, wl_ref[...], preferred_element_type=jnp.float32)
    acc = acc + jnp.dot(h_ref[...], wr_ref[...],
                        preferred_element_type=jnp.float32)
    hb = jnp.maximum(acc + bl_ref[...], 0.0)
    bfull = jnp.broadcast_to(b_ref[...], (rb, hd))
    gid = lax.broadcasted_iota(jnp.int32, (rb, g), 1)
    mask = (b_ref[...] == gid).astype(jnp.float32)
    dn = (((0,), (0,)), ((), ()))
    sums[...] += lax.dot_general(mask, hb, dn,
                                 preferred_element_type=jnp.float32)
    cnts[...] += lax.dot_general(mask, jnp.ones_like(hb), dn,
                                 preferred_element_type=jnp.float32)
    bmin = b_ref[0, 0]
    bmax = b_ref[rb - 1, 0]

    def gbody(gg, carry):
      m = jnp.max(jnp.where(bfull == gg, hb, 0.0), axis=0, keepdims=True)
      maxs[pl.ds(gg, 1), :] = jnp.maximum(maxs[pl.ds(gg, 1), :], m)
      return carry

    lax.fori_loop(bmin, bmax + 1, gbody, 0)

    @pl.when(i == nb - 1)
    def _final():
      meanp = sums[...] / jnp.maximum(cnts[...], 1.0)
      z1 = jnp.dot(meanp, w1a_ref[...], preferred_element_type=jnp.float32)
      z1 = z1 + jnp.dot(maxs[...], w1b_ref[...],
                        preferred_element_type=jnp.float32)
      z1 = z1 + jnp.dot(gf_ref[...], w1c_ref[...],
                        preferred_element_type=jnp.float32)
      z1 = jnp.maximum(z1 + b1_ref[...], 0.0)
      z2 = jnp.dot(z1, w2_ref[...], preferred_element_type=jnp.float32)
      o_ref[...] = jnp.maximum(z2 + b2_ref[...], 0.0)

  return pl.pallas_call(
      body,
      grid=(nb,),
      in_specs=[
          pl.BlockSpec((NC, rb, aw), lambda i: (0, i, 0)),
          pl.BlockSpec((NC, rb, CW), lambda i: (0, i, 0)),
          pl.BlockSpec((rb, hd), lambda i: (i, 0)),
          pl.BlockSpec((hd, hd), lambda i: (0, 0)),
          pl.BlockSpec((1, hd), lambda i: (0, 0)),
          pl.BlockSpec((hd, hd), lambda i: (0, 0)),
          pl.BlockSpec((rb, 1), lambda i: (i, 0)),
          pl.BlockSpec((g, gfd), lambda i: (0, 0)),
          pl.BlockSpec((hd, hd), lambda i: (0, 0)),
          pl.BlockSpec((hd, hd), lambda i: (0, 0)),
          pl.BlockSpec((gfd, hd), lambda i: (0, 0)),
          pl.BlockSpec((1, hd), lambda i: (0, 0)),
          pl.BlockSpec((hd, hd), lambda i: (0, 0)),
          pl.BlockSpec((1, hd), lambda i: (0, 0)),
      ],
      out_specs=pl.BlockSpec((g, hd), lambda i: (0, 0)),
      out_shape=jax.ShapeDtypeStruct((g, hd), jnp.float32),
      scratch_shapes=[
          pltpu.VMEM((g, hd), jnp.float32),
          pltpu.VMEM((g, hd), jnp.float32),
          pltpu.VMEM((g, hd), jnp.float32),
      ],
  )(agg2, cnt2, h, wl, bl.reshape(1, hd), wr, batch_col, gf,
    w1a, w1b, w1c, b1.reshape(1, hd), w2, b2.reshape(1, hd))


def kernel(x, edge_index, batch, graph_feat, W_in, b_in,
           Wl0, bl0, Wr0, Wl1, bl1, Wr1, Wl2, bl2, Wr2,
           W1, b1, W2, b2):
  n = x.shape[0]
  hd = W_in.shape[1]
  e = edge_index.shape[1]
  nw = NC * NS
  edges = edge_index.reshape(2, nw, -1, 80)

  z64 = jnp.zeros((n, hd), jnp.float32)
  z16 = jnp.zeros((n, CW), jnp.float32)
  ones16 = jnp.ones((80, CW), jnp.float32)

  seg_cnt = _make_sc_segsum(n, hd, e, True)
  seg_nc = _make_sc_segsum(n, hd, e, False)

  h = _input_proj(x, W_in, b_in)
  agg2 = seg_cnt(h, edges, z64, z16, ones16)
  cnt2 = lax.slice(agg2, (0, 0, hd), (NC, n, hd + CW))
  h = _layer_combine(agg2, None, h, Wl0, bl0, Wr0)
  agg2 = seg_nc(h, edges, z64)
  h = _layer_combine(agg2, cnt2, h, Wl1, bl1, Wr1)
  agg2 = seg_nc(h, edges, z64)
  return _combine_pool(agg2, cnt2, h, Wl2, bl2, Wr2,
                       batch.reshape(n, 1), graph_feat, W1, b1, W2, b2)


# speedup vs baseline: 16.2209x; 1.0020x over previous
import functools

import jax
import jax.numpy as jnp
from jax import lax
from jax.experimental import pallas as pl
from jax.experimental.pallas import tpu as pltpu
from jax.experimental.pallas import tpu_sc as plsc

NC = 2
NS = 16
CW = 16


@functools.lru_cache(maxsize=None)
def _make_sc_segsum(n_nodes: int, h_dim: int, n_edges: int,
                    with_cnt: bool = False):
  NW = NC * NS
  epw = n_edges // NW
  K = 80
  nch = epw // K
  GR = 5
  ngr = nch // GR
  assert epw % K == 0 and nch % GR == 0
  rps = (n_nodes // NS) // 8 * 8
  tail = n_nodes - NS * rps

  mesh = plsc.VectorSubcoreMesh(core_axis_name="c", subcore_axis_name="s")
  params = pltpu.CompilerParams(use_tc_tiling_on_sc=False)

  OW = 128
  agg_ty = jax.ShapeDtypeStruct((NC, n_nodes, OW), jnp.float32)
  scratch = [
      pltpu.VMEM_SHARED((n_nodes, h_dim), jnp.float32),
      pltpu.VMEM((nch, K), jnp.int32),
      pltpu.VMEM((nch, K), jnp.int32),
      pltpu.VMEM((2 * GR * K, h_dim), jnp.float32),
      pltpu.SemaphoreType.DMA,
      pltpu.SemaphoreType.DMA,
      pltpu.SemaphoreType.DMA,
  ]
  if with_cnt:
    scratch += [
        pltpu.VMEM_SHARED((n_nodes, CW), jnp.float32),
        pltpu.VMEM((K, CW), jnp.float32),
        pltpu.SemaphoreType.DMA,
    ]

  @functools.partial(
      pl.kernel,
      out_type=agg_ty,
      mesh=mesh,
      compiler_params=params,
      scratch_types=scratch,
  )
  def seg(*refs):
    if with_cnt:
      (h_hbm, e_hbm, z_hbm, z16_hbm, ones_hbm, agg_out,
       agg_s, src_v, dst_v, rows_v, gsem, ssem_a, ssem_b,
       cnt_s, ones_v, csem) = refs
    else:
      (h_hbm, e_hbm, z_hbm, agg_out,
       agg_s, src_v, dst_v, rows_v, gsem, ssem_a, ssem_b) = refs
    c = lax.axis_index("c")
    s = lax.axis_index("s")
    r0 = s * rps
    pltpu.sync_copy(z_hbm.at[pl.ds(r0, rps)], agg_s.at[pl.ds(r0, rps)])
    if with_cnt:
      pltpu.sync_copy(z16_hbm.at[pl.ds(r0, rps)], cnt_s.at[pl.ds(r0, rps)])
      pltpu.sync_copy(ones_hbm, ones_v)
    if tail:
      @pl.when(s == 0)
      def _tail_init():
        t0 = NS * rps
        pltpu.sync_copy(z_hbm.at[pl.ds(t0, tail)], agg_s.at[pl.ds(t0, tail)])
        if with_cnt:
          pltpu.sync_copy(z16_hbm.at[pl.ds(t0, tail)],
                          cnt_s.at[pl.ds(t0, tail)])
    w = c * NS + s
    pltpu.sync_copy(e_hbm.at[0, w], src_v)
    pltpu.sync_copy(e_hbm.at[1, w], dst_v)
    plsc.subcore_barrier()

    def fire_g(base_ch, half):
      return [pltpu.async_copy(
          h_hbm.at[src_v.at[base_ch + b]],
          rows_v.at[pl.ds((half * GR + b) * K, K)], gsem)
          for b in range(GR)]

    def fire_s(base_ch, half, sem):
      ds = [pltpu.async_copy(
          rows_v.at[pl.ds((half * GR + b) * K, K)],
          agg_s.at[dst_v.at[base_ch + b]], sem, add=True)
          for b in range(GR)]
      if with_cnt:
        ds += [pltpu.async_copy(ones_v, cnt_s.at[dst_v.at[base_ch + b]],
                                csem, add=True)
               for b in range(GR)]
      return ds

    def drain_s(sem):
      pltpu.make_async_copy(
          h_hbm.at[pl.ds(0, GR * K)], rows_v.at[pl.ds(0, GR * K)], sem).wait()

    def drain_c():
      if with_cnt:
        for _ in range(GR):
          pltpu.make_async_copy(z16_hbm.at[pl.ds(0, K)], ones_v, csem).wait()

    def body2(g2, carry):
      ca = g2 * 2 * GR
      cb = ca + GR

      @pl.when(g2 > 0)
      def _da():
        drain_s(ssem_a)
        drain_c()

      ga = fire_g(ca, 0)

      @pl.when(g2 > 0)
      def _db():
        drain_s(ssem_b)
        drain_c()

      gb = fire_g(cb, 1)
      for d in ga:
        d.wait()
      fire_s(ca, 0, ssem_a)
      for d in gb:
        d.wait()
      fire_s(cb, 1, ssem_b)
      return carry

    lax.fori_loop(0, ngr // 2, body2, 0)
    drain_s(ssem_a)
    drain_s(ssem_b)
    drain_c()
    drain_c()
    if ngr % 2:
      ct = (ngr - 1) * GR
      for d in fire_g(ct, 0):
        d.wait()
      for d in fire_s(ct, 0, ssem_a):
        d.wait()

    plsc.subcore_barrier()

    def wb(rlo, nrows):
      pltpu.sync_copy(agg_s.at[pl.ds(rlo, nrows)],
                      agg_out.at[c, pl.ds(rlo, nrows), pl.ds(0, h_dim)])
      if with_cnt:
        pltpu.sync_copy(cnt_s.at[pl.ds(rlo, nrows)],
                        agg_out.at[c, pl.ds(rlo, nrows), pl.ds(h_dim, CW)])

    wb(r0, rps)
    if tail:
      @pl.when(s == 0)
      def _tail_out():
        wb(NS * rps, tail)

  return seg


def _input_proj(x, w, b):
  n, d = x.shape
  h = w.shape[1]
  rb = 2000

  def body(x_ref, w_ref, b_ref, o_ref):
    acc = jnp.dot(x_ref[...], w_ref[...], preferred_element_type=jnp.float32)
    o_ref[...] = jnp.maximum(acc + b_ref[...], 0.0)

  return pl.pallas_call(
      body,
      grid=(n // rb,),
      in_specs=[
          pl.BlockSpec((rb, d), lambda i: (i, 0)),
          pl.BlockSpec((d, h), lambda i: (0, 0)),
          pl.BlockSpec((1, h), lambda i: (0, 0)),
      ],
      out_specs=pl.BlockSpec((rb, h), lambda i: (i, 0)),
      out_shape=jax.ShapeDtypeStruct((n, h), jnp.float32),
  )(x, w, b.reshape(1, h))


def _layer_combine(agg2, cnt2, h, wl, bl, wr):
  n = h.shape[0]
  hw = h.shape[1]
  aw = agg2.shape[2]
  hd = wl.shape[0]
  rb = 2000
  aug = cnt2 is None

  def body(*refs):
    if aug:
      a_ref, h_ref, wl_ref, bl_ref, wr_ref, o_ref = refs
    else:
      a_ref, c_ref, h_ref, wl_ref, bl_ref, wr_ref, o_ref = refs
    a = a_ref[0, :, :hd] + a_ref[1, :, :hd]
    if aug:
      cg = a_ref[0, :, hd:hd + 1] + a_ref[1, :, hd:hd + 1]
    else:
      cg = c_ref[0, :, :1] + c_ref[1, :, :1]
    mean = a * (1.0 / jnp.maximum(cg, 1.0))
    acc = jnp.dot(mean, wl_ref[...], preferred_element_type=jnp.float32)
    acc = acc + jnp.dot(h_ref[:, :hd], wr_ref[...],
                        preferred_element_type=jnp.float32)
    o_ref[...] = jnp.maximum(acc + bl_ref[...], 0.0)

  in_specs = [pl.BlockSpec((NC, rb, aw), lambda i: (0, i, 0))]
  args = [agg2]
  if not aug:
    in_specs.append(pl.BlockSpec((NC, rb, CW), lambda i: (0, i, 0)))
    args.append(cnt2)
  in_specs += [
      pl.BlockSpec((rb, hw), lambda i: (i, 0)),
      pl.BlockSpec((hd, hd), lambda i: (0, 0)),
      pl.BlockSpec((1, hd), lambda i: (0, 0)),
      pl.BlockSpec((hd, hd), lambda i: (0, 0)),
  ]
  args += [h, wl, bl.reshape(1, hd), wr]

  return pl.pallas_call(
      body,
      grid=(n // rb,),
      in_specs=in_specs,
      out_specs=pl.BlockSpec((rb, hd), lambda i: (i, 0)),
      out_shape=jax.ShapeDtypeStruct((n, hd), jnp.float32),
  )(*args)


def _combine_pool(agg2, cnt2, h, wl, bl, wr, batch_col, gf, w1, b1, w2, b2):
  n, hd = h.shape
  aw = agg2.shape[2]
  g = gf.shape[0]
  gfd = gf.shape[1]
  rb = 1000
  nb = n // rb
  w1a = w1[:hd]
  w1b = w1[hd:2 * hd]
  w1c = w1[2 * hd:]

  def body(a_ref, c_ref, h_ref, wl_ref, bl_ref, wr_ref,
           b_ref, gf_ref, w1a_ref, w1b_ref, w1c_ref, b1_ref,
           w2_ref, b2_ref, o_ref, sums, cnts, maxs):
    i = pl.program_id(0)

    @pl.when(i == 0)
    def _init():
      sums[...] = jnp.zeros_like(sums)
      cnts[...] = jnp.zeros_like(cnts)
      maxs[...] = jnp.zeros_like(maxs)

    a = a_ref[0, :, :hd] + a_ref[1, :, :hd]
    cg = c_ref[0, :, :1] + c_ref[1, :, :1]
    mean = a * (1.0 / jnp.maximum(cg, 1.0))
    acc = jnp.dot(mean, wl_ref[...], preferred_element_type=jnp.float32)
    acc = acc + jnp.dot(h_ref[...], wr_ref[...],
                        preferred_element_type=jnp.float32)
    hb = jnp.maximum(acc + bl_ref[...], 0.0)
    bfull = jnp.broadcast_to(b_ref[...], (rb, hd))
    gid = lax.broadcasted_iota(jnp.int32, (rb, g), 1)
    mask = (b_ref[...] == gid).astype(jnp.float32)
    dn = (((0,), (0,)), ((), ()))
    sums[...] += lax.dot_general(mask, hb, dn,
                                 preferred_element_type=jnp.float32)
    cnts[...] += lax.dot_general(mask, jnp.ones_like(hb), dn,
                                 preferred_element_type=jnp.float32)
    bmin = b_ref[0, 0]
    bmax = b_ref[rb - 1, 0]

    def gbody(gg, carry):
      m = jnp.max(jnp.where(bfull == gg, hb, 0.0), axis=0, keepdims=True)
      maxs[pl.ds(gg, 1), :] = jnp.maximum(maxs[pl.ds(gg, 1), :], m)
      return carry

    lax.fori_loop(bmin, bmax + 1, gbody, 0)

    @pl.when(i == nb - 1)
    def _final():
      meanp = sums[...] / jnp.maximum(cnts[...], 1.0)
      z1 = jnp.dot(meanp, w1a_ref[...], preferred_element_type=jnp.float32)
      z1 = z1 + jnp.dot(maxs[...], w1b_ref[...],
                        preferred_element_type=jnp.float32)
      z1 = z1 + jnp.dot(gf_ref[...], w1c_ref[...],
                        preferred_element_type=jnp.float32)
      z1 = jnp.maximum(z1 + b1_ref[...], 0.0)
      z2 = jnp.dot(z1, w2_ref[...], preferred_element_type=jnp.float32)
      o_ref[...] = jnp.maximum(z2 + b2_ref[...], 0.0)

  return pl.pallas_call(
      body,
      grid=(nb,),
      in_specs=[
          pl.BlockSpec((NC, rb, aw), lambda i: (0, i, 0)),
          pl.BlockSpec((NC, rb, CW), lambda i: (0, i, 0)),
          pl.BlockSpec((rb, hd), lambda i: (i, 0)),
          pl.BlockSpec((hd, hd), lambda i: (0, 0)),
          pl.BlockSpec((1, hd), lambda i: (0, 0)),
          pl.BlockSpec((hd, hd), lambda i: (0, 0)),
          pl.BlockSpec((rb, 1), lambda i: (i, 0)),
          pl.BlockSpec((g, gfd), lambda i: (0, 0)),
          pl.BlockSpec((hd, hd), lambda i: (0, 0)),
          pl.BlockSpec((hd, hd), lambda i: (0, 0)),
          pl.BlockSpec((gfd, hd), lambda i: (0, 0)),
          pl.BlockSpec((1, hd), lambda i: (0, 0)),
          pl.BlockSpec((hd, hd), lambda i: (0, 0)),
          pl.BlockSpec((1, hd), lambda i: (0, 0)),
      ],
      out_specs=pl.BlockSpec((g, hd), lambda i: (0, 0)),
      out_shape=jax.ShapeDtypeStruct((g, hd), jnp.float32),
      scratch_shapes=[
          pltpu.VMEM((g, hd), jnp.float32),
          pltpu.VMEM((g, hd), jnp.float32),
          pltpu.VMEM((g, hd), jnp.float32),
      ],
  )(agg2, cnt2, h, wl, bl.reshape(1, hd), wr, batch_col, gf,
    w1a, w1b, w1c, b1.reshape(1, hd), w2, b2.reshape(1, hd))


def kernel(x, edge_index, batch, graph_feat, W_in, b_in,
           Wl0, bl0, Wr0, Wl1, bl1, Wr1, Wl2, bl2, Wr2,
           W1, b1, W2, b2):
  n = x.shape[0]
  hd = W_in.shape[1]
  e = edge_index.shape[1]
  nw = NC * NS
  edges = edge_index.reshape(2, nw, -1, 80)

  z64 = jnp.zeros((n, hd), jnp.float32)
  z16 = jnp.zeros((n, CW), jnp.float32)
  ones16 = jnp.ones((80, CW), jnp.float32)

  seg_cnt = _make_sc_segsum(n, hd, e, True)
  seg_nc = _make_sc_segsum(n, hd, e, False)

  h = _input_proj(x, W_in, b_in)
  agg2 = seg_cnt(h, edges, z64, z16, ones16)
  cnt2 = lax.slice(agg2, (0, 0, hd), (NC, n, hd + CW))
  h = _layer_combine(agg2, None, h, Wl0, bl0, Wr0)
  agg2 = seg_nc(h, edges, z64)
  h = _layer_combine(agg2, cnt2, h, Wl1, bl1, Wr1)
  agg2 = seg_nc(h, edges, z64)
  return _combine_pool(agg2, cnt2, h, Wl2, bl2, Wr2,
                       batch.reshape(n, 1), graph_feat, W1, b1, W2, b2)
